# Initial kernel scaffold; baseline (speedup 1.0000x reference)
import jax
import jax.numpy as jnp
from jax.experimental import pallas as pl


def kernel(features, positions, edge_index, W1, b1, W2, b2, W3, b3):
    raise NotImplementedError("write your pallas kernel here")


# speedup vs baseline: 3.0242x; 3.0242x over previous
import functools
import math

import jax
import jax.numpy as jnp
from jax import lax
from jax.experimental import pallas as pl
from jax.experimental.pallas import tpu as pltpu
from jax.experimental.pallas import tpu_sc as plsc

N_NODES = 10000
N_EDGES = 320000
IN_C = 128
OUT_C = 128
HID = 32
N_SH = 9

POS_W = 16
TB_W = POS_W + HID
MSG_W = 48

NC = 2
NS = 16
NW = NC * NS
E_PER_W = N_EDGES // NW
CHUNK = 128
NFULL = E_PER_W // CHUNK
TAIL = E_PER_W - NFULL * CHUNK
ROWS_PER_SUB = N_NODES // NS

_F32 = jnp.float32


def _silu(x):
    return x * (1.0 / (1.0 + jnp.exp(-x)))


def _prep_body(pos16_ref, feat_ref, w1b_ref, b1_ref, tb_ref):
    f1 = jnp.dot(feat_ref[...], w1b_ref[...], preferred_element_type=_F32)
    tb_ref[:, 0:POS_W] = pos16_ref[...]
    tb_ref[:, POS_W:TB_W] = f1 + b1_ref[...]


def _prep(pos16, features, W1b, b1r):
    blk = 2000
    grid = N_NODES // blk
    return pl.pallas_call(
        _prep_body,
        grid=(grid,),
        in_specs=[
            pl.BlockSpec((blk, POS_W), lambda i: (i, 0)),
            pl.BlockSpec((blk, IN_C), lambda i: (i, 0)),
            pl.BlockSpec((IN_C, HID), lambda i: (0, 0)),
            pl.BlockSpec((1, HID), lambda i: (0, 0)),
        ],
        out_specs=pl.BlockSpec((blk, TB_W), lambda i: (i, 0)),
        out_shape=jax.ShapeDtypeStruct((N_NODES, TB_W), _F32),
    )(pos16, features, W1b, b1r)


def _gather_kernel_body(pos16_hbm, tableb_hbm, row_hbm, col_hbm,
                        posr_out, pcf_out,
                        rbuf, cbuf, rtail, ctail, gr, gc, grt, gct,
                        sem1, sem2):
    wid = lax.axis_index("s") * NC + lax.axis_index("c")
    base = wid * E_PER_W

    @pl.loop(0, NFULL)
    def _(i):
        off = base + i * CHUNK
        d1 = pltpu.async_copy(row_hbm.at[pl.ds(off, CHUNK)], rbuf, sem1)
        d2 = pltpu.async_copy(col_hbm.at[pl.ds(off, CHUNK)], cbuf, sem2)
        d1.wait()
        d2.wait()
        g1 = pltpu.async_copy(pos16_hbm.at[rbuf], gr, sem1)
        g2 = pltpu.async_copy(tableb_hbm.at[cbuf], gc, sem2)
        g1.wait()
        g2.wait()
        pltpu.sync_copy(gr, posr_out.at[pl.ds(off, CHUNK)])
        pltpu.sync_copy(gc, pcf_out.at[pl.ds(off, CHUNK)])

    off = base + NFULL * CHUNK
    d1 = pltpu.async_copy(row_hbm.at[pl.ds(off, TAIL)], rtail, sem1)
    d2 = pltpu.async_copy(col_hbm.at[pl.ds(off, TAIL)], ctail, sem2)
    d1.wait()
    d2.wait()
    g1 = pltpu.async_copy(pos16_hbm.at[rtail], grt, sem1)
    g2 = pltpu.async_copy(tableb_hbm.at[ctail], gct, sem2)
    g1.wait()
    g2.wait()
    pltpu.sync_copy(grt, posr_out.at[pl.ds(off, TAIL)])
    pltpu.sync_copy(gct, pcf_out.at[pl.ds(off, TAIL)])


def _sc_gather(pos16, tableb, row, col):
    mesh = plsc.VectorSubcoreMesh(core_axis_name="c", subcore_axis_name="s",
                                  num_cores=NC, num_subcores=NS)
    fn = pl.kernel(
        _gather_kernel_body,
        out_type=(jax.ShapeDtypeStruct((N_EDGES, POS_W), _F32),
                  jax.ShapeDtypeStruct((N_EDGES, TB_W), _F32)),
        mesh=mesh,
        scratch_types=[
            pltpu.VMEM((CHUNK,), jnp.int32),
            pltpu.VMEM((CHUNK,), jnp.int32),
            pltpu.VMEM((TAIL,), jnp.int32),
            pltpu.VMEM((TAIL,), jnp.int32),
            pltpu.VMEM((CHUNK, POS_W), _F32),
            pltpu.VMEM((CHUNK, TB_W), _F32),
            pltpu.VMEM((TAIL, POS_W), _F32),
            pltpu.VMEM((TAIL, TB_W), _F32),
            pltpu.SemaphoreType.DMA,
            pltpu.SemaphoreType.DMA,
        ],
        compiler_params=pltpu.CompilerParams(use_tc_tiling_on_sc=False),
    )
    return fn(pos16, tableb, row, col)


_SH_C0 = 0.5 / math.sqrt(math.pi)
_SH_C1 = 0.5 * math.sqrt(3.0 / (2.0 * math.pi))
_SH_C2 = 0.5 * math.sqrt(3.0 / math.pi)
_SH_C4 = 0.25 * math.sqrt(15.0 / (2.0 * math.pi))
_SH_C5 = 0.5 * math.sqrt(15.0 / (2.0 * math.pi))
_SH_C6 = 0.25 * math.sqrt(5.0 / math.pi)


def _mlp_body(posr_ref, pcf_ref, w1a_ref, w2_ref, b2_ref, out_ref):
    pr = posr_ref[...]
    pc = pcf_ref[...]
    rx = pr[:, 0:1] - pc[:, 0:1]
    ry = pr[:, 1:2] - pc[:, 1:2]
    rz = pr[:, 2:3] - pc[:, 2:3]
    n1 = jnp.maximum(jnp.sqrt(rx * rx + ry * ry + rz * rz), 1e-8)
    inv = 1.0 / n1
    x = rx * inv
    y = ry * inv
    z = rz * inv
    x2y2 = x * x - y * y
    zx = z * x
    ys = (
        jnp.full_like(x, _SH_C0),
        -_SH_C1 * x,
        _SH_C2 * z,
        _SH_C1 * x,
        _SH_C4 * x2y2,
        -_SH_C5 * zx,
        _SH_C6 * (2.0 * z * z - x * x - y * y),
        _SH_C5 * zx,
        _SH_C4 * x2y2,
    )
    t = pc[:, POS_W:TB_W]
    for k in range(N_SH):
        t = t + ys[k] * w1a_ref[k:k + 1, :]
    h1 = _silu(t)
    h2 = _silu(jnp.dot(h1, w2_ref[...], preferred_element_type=_F32) + b2_ref[...])
    out_ref[:, 0:HID] = h2
    out_ref[:, HID:HID + 1] = jnp.ones_like(x)
    out_ref[:, HID + 1:MSG_W] = jnp.zeros((pr.shape[0], MSG_W - HID - 1), _F32)


def _mlp(posr, pcf, W1a16, W2, b2r):
    blk = 2000
    grid = N_EDGES // blk
    return pl.pallas_call(
        _mlp_body,
        grid=(grid,),
        in_specs=[
            pl.BlockSpec((blk, POS_W), lambda i: (i, 0)),
            pl.BlockSpec((blk, TB_W), lambda i: (i, 0)),
            pl.BlockSpec((16, HID), lambda i: (0, 0)),
            pl.BlockSpec((HID, HID), lambda i: (0, 0)),
            pl.BlockSpec((1, HID), lambda i: (0, 0)),
        ],
        out_specs=pl.BlockSpec((blk, MSG_W), lambda i: (i, 0)),
        out_shape=jax.ShapeDtypeStruct((N_EDGES, MSG_W), _F32),
    )(posr, pcf, W1a16, W2, b2r)


def _scatter_kernel_body(h2_hbm, row_hbm, zeros_hbm, partials_out,
                         rbuf, rtail, mbuf, mtail, acc, sem1, sem2):
    c = lax.axis_index("c")
    s = lax.axis_index("s")
    wid = s * NC + c
    pltpu.sync_copy(zeros_hbm.at[pl.ds(s * ROWS_PER_SUB, ROWS_PER_SUB)],
                    acc.at[pl.ds(s * ROWS_PER_SUB, ROWS_PER_SUB)])
    plsc.subcore_barrier()

    base = wid * E_PER_W

    @pl.loop(0, NFULL)
    def _(i):
        off = base + i * CHUNK
        d1 = pltpu.async_copy(row_hbm.at[pl.ds(off, CHUNK)], rbuf, sem1)
        d2 = pltpu.async_copy(h2_hbm.at[pl.ds(off, CHUNK)], mbuf, sem2)
        d1.wait()
        d2.wait()
        pltpu.sync_copy(mbuf, acc.at[rbuf], add=True)

    off = base + NFULL * CHUNK
    d1 = pltpu.async_copy(row_hbm.at[pl.ds(off, TAIL)], rtail, sem1)
    d2 = pltpu.async_copy(h2_hbm.at[pl.ds(off, TAIL)], mtail, sem2)
    d1.wait()
    d2.wait()
    pltpu.sync_copy(mtail, acc.at[rtail], add=True)

    plsc.subcore_barrier()
    pltpu.sync_copy(acc.at[pl.ds(s * ROWS_PER_SUB, ROWS_PER_SUB)],
                    partials_out.at[c].at[pl.ds(s * ROWS_PER_SUB, ROWS_PER_SUB)])


def _sc_scatter(h2ext, row, zeros_nm):
    mesh = plsc.VectorSubcoreMesh(core_axis_name="c", subcore_axis_name="s",
                                  num_cores=NC, num_subcores=NS)
    fn = pl.kernel(
        _scatter_kernel_body,
        out_type=jax.ShapeDtypeStruct((NC, N_NODES, MSG_W), _F32),
        mesh=mesh,
        scratch_types=[
            pltpu.VMEM((CHUNK,), jnp.int32),
            pltpu.VMEM((TAIL,), jnp.int32),
            pltpu.VMEM((CHUNK, MSG_W), _F32),
            pltpu.VMEM((TAIL, MSG_W), _F32),
            pltpu.VMEM_SHARED((N_NODES, MSG_W), _F32),
            pltpu.SemaphoreType.DMA,
            pltpu.SemaphoreType.DMA,
        ],
        compiler_params=pltpu.CompilerParams(use_tc_tiling_on_sc=False),
    )
    return fn(h2ext, row, zeros_nm)


def _final_body(p_ref, w3_ref, b3r_ref, out_ref):
    p = p_ref[0] + p_ref[1]
    w3s = w3_ref[:, 0:OUT_C]
    for k in range(1, N_SH):
        w3s = w3s + w3_ref[:, k * OUT_C:(k + 1) * OUT_C]
    b3s = jnp.sum(b3r_ref[...], axis=0, keepdims=True)
    out_ref[...] = (jnp.dot(p[:, 0:HID], w3s, preferred_element_type=_F32)
                    + p[:, HID:HID + 1] * b3s)


def _final(partials, W3, b3r16):
    blk = 2000
    grid = N_NODES // blk
    return pl.pallas_call(
        _final_body,
        grid=(grid,),
        in_specs=[
            pl.BlockSpec((NC, blk, MSG_W), lambda i: (0, i, 0)),
            pl.BlockSpec((HID, N_SH * OUT_C), lambda i: (0, 0)),
            pl.BlockSpec((16, OUT_C), lambda i: (0, 0)),
        ],
        out_specs=pl.BlockSpec((blk, OUT_C), lambda i: (i, 0)),
        out_shape=jax.ShapeDtypeStruct((N_NODES, OUT_C), _F32),
    )(partials, W3, b3r16)


def kernel(features, positions, edge_index, W1, b1, W2, b2, W3, b3):
    ei = edge_index.astype(jnp.int32)
    row = ei[0]
    col = ei[1]

    pos16 = jnp.pad(positions.astype(_F32), ((0, 0), (0, POS_W - 3)))
    W1a16 = jnp.pad(W1[:N_SH], ((0, 16 - N_SH), (0, 0)))
    W1b = W1[N_SH:]
    b1r = b1.reshape(1, HID)
    b2r = b2.reshape(1, HID)
    b3r16 = jnp.pad(b3.reshape(N_SH, OUT_C), ((0, 16 - N_SH), (0, 0)))
    zeros_nm = jnp.zeros((N_NODES, MSG_W), _F32)

    tableb = _prep(pos16, features, W1b, b1r)
    posr, pcf = _sc_gather(pos16, tableb, row, col)
    h2ext = _mlp(posr, pcf, W1a16, W2, b2r)
    partials = _sc_scatter(h2ext, row, zeros_nm)
    return _final(partials, W3, b3r16)


# speedup vs baseline: 12.8369x; 4.2447x over previous
import functools
import math

import jax
import jax.numpy as jnp
from jax import lax
from jax.experimental import pallas as pl
from jax.experimental.pallas import tpu as pltpu
from jax.experimental.pallas import tpu_sc as plsc

N_NODES = 10000
N_EDGES = 320000
IN_C = 128
OUT_C = 128
HID = 32
N_SH = 9

POS_W = 16
S_W = 8
PACK = 4
DEG_W = 16

NC = 2
NS = 16
NW = NC * NS
E_PER_W = N_EDGES // NW
CHUNK = 128
NFULL = E_PER_W // CHUNK
TAIL = E_PER_W - NFULL * CHUNK
ROWS_PER_SUB = N_NODES // NS
LANES = 16

_F32 = jnp.float32


def _silu(x):
    return x * (1.0 / (1.0 + jnp.exp(-x)))


def _prep_body(feat_ref, w1b_ref, b1_ref, f1_ref):
    f1_ref[...] = (jnp.dot(feat_ref[...], w1b_ref[...],
                           preferred_element_type=_F32) + b1_ref[...])


def _prep(features, W1b, b1e):
    blk = 2000
    grid = N_NODES // blk
    return pl.pallas_call(
        _prep_body,
        grid=(grid,),
        in_specs=[
            pl.BlockSpec((blk, IN_C), lambda i: (i, 0)),
            pl.BlockSpec((IN_C, HID), lambda i: (0, 0)),
            pl.BlockSpec((1, HID), lambda i: (0, 0)),
        ],
        out_specs=pl.BlockSpec((blk, HID), lambda i: (i, 0)),
        out_shape=jax.ShapeDtypeStruct((N_NODES, HID), _F32),
    )(features, W1b, b1e)


def _rsqrt_sc(n):
    i = plsc.bitcast(n, jnp.int32)
    i = jnp.full_like(i, 0x5F3759DF) - lax.shift_right_logical(i, 1)
    y = plsc.bitcast(i, _F32)
    y = y * (1.5 - 0.5 * n * y * y)
    y = y * (1.5 - 0.5 * n * y * y)
    return y


def _geom_chunk(k, gr, gc, gf, sbuf, f1buf):
    del gf, f1buf
    base_iota = lax.iota(jnp.int32, LANES)

    @pl.loop(0, k // LANES)
    def _(g):
        ridx = base_iota + g * LANES
        c0 = jnp.zeros((LANES,), jnp.int32)
        c1 = jnp.full((LANES,), 1, jnp.int32)
        c2 = jnp.full((LANES,), 2, jnp.int32)
        rx = (plsc.load_gather(gr, [ridx, c0]) - plsc.load_gather(gc, [ridx, c0]))
        ry = (plsc.load_gather(gr, [ridx, c1]) - plsc.load_gather(gc, [ridx, c1]))
        rz = (plsc.load_gather(gr, [ridx, c2]) - plsc.load_gather(gc, [ridx, c2]))
        n = jnp.maximum(rx * rx + ry * ry + rz * rz, 1e-16)
        inv = _rsqrt_sc(n)
        x = rx * inv
        y = ry * inv
        z = rz * inv
        x2 = x * x
        y2 = y * y
        z2 = z * z
        xz = x * z
        cols = (x, y, z, x2, y2, z2, xz, x)
        for j in range(S_W):
            plsc.store_scatter(sbuf, [ridx, jnp.full((LANES,), j, jnp.int32)],
                               cols[j])


def _gather_kernel_body(pos16_hbm, f1_hbm, row_hbm, col_hbm,
                        s_out, f1c_out,
                        rbuf, cbuf, rtail, ctail,
                        gr, gc, gf, grt, gct, gft,
                        sbuf, stail,
                        sem1, sem2, sem3):
    wid = lax.axis_index("s") * NC + lax.axis_index("c")
    base = wid * E_PER_W

    @pl.loop(0, NFULL)
    def _(i):
        off = base + i * CHUNK
        d1 = pltpu.async_copy(row_hbm.at[pl.ds(off, CHUNK)], rbuf, sem1)
        d2 = pltpu.async_copy(col_hbm.at[pl.ds(off, CHUNK)], cbuf, sem2)
        d1.wait()
        d2.wait()
        g1 = pltpu.async_copy(pos16_hbm.at[rbuf], gr, sem1)
        g2 = pltpu.async_copy(pos16_hbm.at[cbuf], gc, sem2)
        g3 = pltpu.async_copy(f1_hbm.at[cbuf], gf, sem3)
        g1.wait()
        g2.wait()
        _geom_chunk(CHUNK, gr, gc, gf, sbuf, None)
        g3.wait()
        w1 = pltpu.async_copy(sbuf, s_out.at[pl.ds(off, CHUNK)], sem1)
        w2 = pltpu.async_copy(gf, f1c_out.at[pl.ds(off, CHUNK)], sem2)
        w1.wait()
        w2.wait()

    off = base + NFULL * CHUNK
    d1 = pltpu.async_copy(row_hbm.at[pl.ds(off, TAIL)], rtail, sem1)
    d2 = pltpu.async_copy(col_hbm.at[pl.ds(off, TAIL)], ctail, sem2)
    d1.wait()
    d2.wait()
    g1 = pltpu.async_copy(pos16_hbm.at[rtail], grt, sem1)
    g2 = pltpu.async_copy(pos16_hbm.at[ctail], gct, sem2)
    g3 = pltpu.async_copy(f1_hbm.at[ctail], gft, sem3)
    g1.wait()
    g2.wait()
    _geom_chunk(TAIL, grt, gct, gft, stail, None)
    g3.wait()
    w1 = pltpu.async_copy(stail, s_out.at[pl.ds(off, TAIL)], sem1)
    w2 = pltpu.async_copy(gft, f1c_out.at[pl.ds(off, TAIL)], sem2)
    w1.wait()
    w2.wait()


def _sc_gather(pos16, f1, row, col):
    mesh = plsc.VectorSubcoreMesh(core_axis_name="c", subcore_axis_name="s",
                                  num_cores=NC, num_subcores=NS)
    fn = pl.kernel(
        _gather_kernel_body,
        out_type=(jax.ShapeDtypeStruct((N_EDGES, S_W), _F32),
                  jax.ShapeDtypeStruct((N_EDGES, HID), _F32)),
        mesh=mesh,
        scratch_types=[
            pltpu.VMEM((CHUNK,), jnp.int32),
            pltpu.VMEM((CHUNK,), jnp.int32),
            pltpu.VMEM((TAIL,), jnp.int32),
            pltpu.VMEM((TAIL,), jnp.int32),
            pltpu.VMEM((CHUNK, POS_W), _F32),
            pltpu.VMEM((CHUNK, POS_W), _F32),
            pltpu.VMEM((CHUNK, HID), _F32),
            pltpu.VMEM((TAIL, POS_W), _F32),
            pltpu.VMEM((TAIL, POS_W), _F32),
            pltpu.VMEM((TAIL, HID), _F32),
            pltpu.VMEM((CHUNK, S_W), _F32),
            pltpu.VMEM((TAIL, S_W), _F32),
            pltpu.SemaphoreType.DMA,
            pltpu.SemaphoreType.DMA,
            pltpu.SemaphoreType.DMA,
        ],
        compiler_params=pltpu.CompilerParams(use_tc_tiling_on_sc=False,
                                             needs_layout_passes=False),
    )
    return fn(pos16, f1, row, col)


def _mlp_body(shat_ref, f1hat_ref, mhat_ref, w2hat_ref, b2hat_ref, out_ref):
    u = (jnp.dot(shat_ref[...], mhat_ref[...], preferred_element_type=_F32)
         + f1hat_ref[...])
    h1 = _silu(u)
    h2 = _silu(jnp.dot(h1, w2hat_ref[...], preferred_element_type=_F32)
               + b2hat_ref[...])
    out_ref[...] = h2


def _mlp(shat, f1hat, Mhat, W2hat, b2hat):
    rows = N_EDGES // PACK
    blk = 2000
    grid = rows // blk
    return pl.pallas_call(
        _mlp_body,
        grid=(grid,),
        in_specs=[
            pl.BlockSpec((blk, PACK * S_W), lambda i: (i, 0)),
            pl.BlockSpec((blk, PACK * HID), lambda i: (i, 0)),
            pl.BlockSpec((PACK * S_W, PACK * HID), lambda i: (0, 0)),
            pl.BlockSpec((PACK * HID, PACK * HID), lambda i: (0, 0)),
            pl.BlockSpec((1, PACK * HID), lambda i: (0, 0)),
        ],
        out_specs=pl.BlockSpec((blk, PACK * HID), lambda i: (i, 0)),
        out_shape=jax.ShapeDtypeStruct((rows, PACK * HID), _F32),
    )(shat, f1hat, Mhat, W2hat, b2hat)


def _scatter_kernel_body(h2_hbm, row_hbm, zeros_h_hbm, zeros_d_hbm, degrows_hbm,
                         ph_out, pd_out,
                         rbuf, rtail, mbuf, mtail, degbuf,
                         acc_h, acc_d, sem1, sem2):
    c = lax.axis_index("c")
    s = lax.axis_index("s")
    wid = s * NC + c
    pltpu.sync_copy(zeros_h_hbm.at[pl.ds(s * ROWS_PER_SUB, ROWS_PER_SUB)],
                    acc_h.at[pl.ds(s * ROWS_PER_SUB, ROWS_PER_SUB)])
    pltpu.sync_copy(zeros_d_hbm.at[pl.ds(s * ROWS_PER_SUB, ROWS_PER_SUB)],
                    acc_d.at[pl.ds(s * ROWS_PER_SUB, ROWS_PER_SUB)])
    pltpu.sync_copy(degrows_hbm, degbuf)
    plsc.subcore_barrier()

    base = wid * E_PER_W

    @pl.loop(0, NFULL)
    def _(i):
        off = base + i * CHUNK
        d1 = pltpu.async_copy(row_hbm.at[pl.ds(off, CHUNK)], rbuf, sem1)
        d2 = pltpu.async_copy(h2_hbm.at[pl.ds(off, CHUNK)], mbuf, sem2)
        d1.wait()
        d2.wait()
        pltpu.sync_copy(mbuf, acc_h.at[rbuf], add=True)
        pltpu.sync_copy(degbuf, acc_d.at[rbuf], add=True)

    off = base + NFULL * CHUNK
    d1 = pltpu.async_copy(row_hbm.at[pl.ds(off, TAIL)], rtail, sem1)
    d2 = pltpu.async_copy(h2_hbm.at[pl.ds(off, TAIL)], mtail, sem2)
    d1.wait()
    d2.wait()
    pltpu.sync_copy(mtail, acc_h.at[rtail], add=True)
    pltpu.sync_copy(degbuf.at[pl.ds(0, TAIL)], acc_d.at[rtail], add=True)

    plsc.subcore_barrier()
    pltpu.sync_copy(acc_h.at[pl.ds(s * ROWS_PER_SUB, ROWS_PER_SUB)],
                    ph_out.at[c].at[pl.ds(s * ROWS_PER_SUB, ROWS_PER_SUB)])
    pltpu.sync_copy(acc_d.at[pl.ds(s * ROWS_PER_SUB, ROWS_PER_SUB)],
                    pd_out.at[c].at[pl.ds(s * ROWS_PER_SUB, ROWS_PER_SUB)])


def _sc_scatter(h2, row, zeros_h, zeros_d, degrows):
    mesh = plsc.VectorSubcoreMesh(core_axis_name="c", subcore_axis_name="s",
                                  num_cores=NC, num_subcores=NS)
    fn = pl.kernel(
        _scatter_kernel_body,
        out_type=(jax.ShapeDtypeStruct((NC, N_NODES, HID), _F32),
                  jax.ShapeDtypeStruct((NC, N_NODES, DEG_W), _F32)),
        mesh=mesh,
        scratch_types=[
            pltpu.VMEM((CHUNK,), jnp.int32),
            pltpu.VMEM((TAIL,), jnp.int32),
            pltpu.VMEM((CHUNK, HID), _F32),
            pltpu.VMEM((TAIL, HID), _F32),
            pltpu.VMEM((CHUNK, DEG_W), _F32),
            pltpu.VMEM_SHARED((N_NODES, HID), _F32),
            pltpu.VMEM_SHARED((N_NODES, DEG_W), _F32),
            pltpu.SemaphoreType.DMA,
            pltpu.SemaphoreType.DMA,
        ],
        compiler_params=pltpu.CompilerParams(use_tc_tiling_on_sc=False),
    )
    return fn(h2, row, zeros_h, zeros_d, degrows)


def _final_body(ph_ref, pd_ref, w3_ref, b3r_ref, out_ref):
    p = ph_ref[0] + ph_ref[1]
    deg = pd_ref[0, :, 0:1] + pd_ref[1, :, 0:1]
    w3s = w3_ref[:, 0:OUT_C]
    for k in range(1, N_SH):
        w3s = w3s + w3_ref[:, k * OUT_C:(k + 1) * OUT_C]
    b3s = jnp.sum(b3r_ref[...], axis=0, keepdims=True)
    out_ref[...] = (jnp.dot(p, w3s, preferred_element_type=_F32) + deg * b3s)


def _final(ph, pd, W3, b3r16):
    blk = 2000
    grid = N_NODES // blk
    return pl.pallas_call(
        _final_body,
        grid=(grid,),
        in_specs=[
            pl.BlockSpec((NC, blk, HID), lambda i: (0, i, 0)),
            pl.BlockSpec((NC, blk, DEG_W), lambda i: (0, i, 0)),
            pl.BlockSpec((HID, N_SH * OUT_C), lambda i: (0, 0)),
            pl.BlockSpec((16, OUT_C), lambda i: (0, 0)),
        ],
        out_specs=pl.BlockSpec((blk, OUT_C), lambda i: (i, 0)),
        out_shape=jax.ShapeDtypeStruct((N_NODES, OUT_C), _F32),
    )(ph, pd, W3, b3r16)


_SH_C0 = 0.5 / math.sqrt(math.pi)
_SH_C1 = 0.5 * math.sqrt(3.0 / (2.0 * math.pi))
_SH_C2 = 0.5 * math.sqrt(3.0 / math.pi)
_SH_C4 = 0.25 * math.sqrt(15.0 / (2.0 * math.pi))
_SH_C5 = 0.5 * math.sqrt(15.0 / (2.0 * math.pi))
_SH_C6 = 0.25 * math.sqrt(5.0 / math.pi)


def _blockdiag(m, k):
    r, c = m.shape
    out = jnp.zeros((k * r, k * c), m.dtype)
    for i in range(k):
        out = out.at[i * r:(i + 1) * r, i * c:(i + 1) * c].set(m)
    return out


def kernel(features, positions, edge_index, W1, b1, W2, b2, W3, b3):
    ei = edge_index.astype(jnp.int32)
    row = ei[0]
    col = ei[1]

    pos16 = jnp.pad(positions.astype(_F32), ((0, 0), (0, POS_W - 3)))
    W1a = W1[:N_SH]
    W1b = W1[N_SH:]
    M = jnp.stack([
        _SH_C1 * (W1a[3] - W1a[1]),
        jnp.zeros((HID,), _F32),
        _SH_C2 * W1a[2],
        _SH_C4 * (W1a[4] + W1a[8]) - _SH_C6 * W1a[6],
        -_SH_C4 * (W1a[4] + W1a[8]) - _SH_C6 * W1a[6],
        2.0 * _SH_C6 * W1a[6],
        _SH_C5 * (W1a[7] - W1a[5]),
        jnp.zeros((HID,), _F32),
    ])
    b1e = (b1 + _SH_C0 * W1a[0]).reshape(1, HID)
    Mhat = _blockdiag(M, PACK)
    W2hat = _blockdiag(W2, PACK)
    b2hat = jnp.tile(b2, (PACK,)).reshape(1, PACK * HID)
    b3r16 = jnp.pad(b3.reshape(N_SH, OUT_C), ((0, 16 - N_SH), (0, 0)))
    zeros_h = jnp.zeros((N_NODES, HID), _F32)
    zeros_d = jnp.zeros((N_NODES, DEG_W), _F32)
    degrows = jnp.zeros((CHUNK, DEG_W), _F32).at[:, 0].set(1.0)

    f1 = _prep(features, W1b, b1e)
    s_mat, f1c = _sc_gather(pos16, f1, row, col)
    shat = s_mat.reshape(N_EDGES // PACK, PACK * S_W)
    f1hat = f1c.reshape(N_EDGES // PACK, PACK * HID)
    h2hat = _mlp(shat, f1hat, Mhat, W2hat, b2hat)
    h2 = h2hat.reshape(N_EDGES, HID)
    ph, pd = _sc_scatter(h2, row, zeros_h, zeros_d, degrows)
    return _final(ph, pd, W3, b3r16)


# speedup vs baseline: 20.3968x; 1.5889x over previous
import functools
import math

import jax
import jax.numpy as jnp
from jax import lax
from jax.experimental import pallas as pl
from jax.experimental.pallas import tpu as pltpu
from jax.experimental.pallas import tpu_sc as plsc

N_NODES = 10000
N_EDGES = 320000
IN_C = 128
OUT_C = 128
HID = 32
N_SH = 9

POS_W = 16
S_W = 8
PACK = 4
DEG_W = 16

NC = 2
NS = 16
NW = NC * NS
E_PER_W = N_EDGES // NW
IDXV = 128
CH = 256
NFULL = E_PER_W // CH
TAIL = E_PER_W - NFULL * CH
ROWS_PER_SUB = N_NODES // NS
LANES = 16

_F32 = jnp.float32


def _silu(x):
    return x * (1.0 / (1.0 + jnp.exp(-x)))


def _prep_body(feat_ref, w1b_ref, b1_ref, f1_ref):
    f1_ref[...] = (jnp.dot(feat_ref[...], w1b_ref[...],
                           preferred_element_type=_F32) + b1_ref[...])


def _prep(features, W1b, b1e):
    blk = 2000
    grid = N_NODES // blk
    return pl.pallas_call(
        _prep_body,
        grid=(grid,),
        in_specs=[
            pl.BlockSpec((blk, IN_C), lambda i: (i, 0)),
            pl.BlockSpec((IN_C, HID), lambda i: (0, 0)),
            pl.BlockSpec((1, HID), lambda i: (0, 0)),
        ],
        out_specs=pl.BlockSpec((blk, HID), lambda i: (i, 0)),
        out_shape=jax.ShapeDtypeStruct((N_NODES, HID), _F32),
    )(features, W1b, b1e)


def _rsqrt_sc(n):
    i = plsc.bitcast(n, jnp.int32)
    i = jnp.full_like(i, 0x5F3759DF) - lax.shift_right_logical(i, 1)
    y = plsc.bitcast(i, _F32)
    y = y * (1.5 - 0.5 * n * y * y)
    y = y * (1.5 - 0.5 * n * y * y)
    return y


def _geom_chunk(k, gr, gc, sbuf):
    base_iota = lax.iota(jnp.int32, LANES)

    @pl.loop(0, k // LANES)
    def _(g):
        ridx = base_iota + g * LANES
        c0 = jnp.zeros((LANES,), jnp.int32)
        c1 = jnp.full((LANES,), 1, jnp.int32)
        c2 = jnp.full((LANES,), 2, jnp.int32)
        rx = (plsc.load_gather(gr, [ridx, c0]) - plsc.load_gather(gc, [ridx, c0]))
        ry = (plsc.load_gather(gr, [ridx, c1]) - plsc.load_gather(gc, [ridx, c1]))
        rz = (plsc.load_gather(gr, [ridx, c2]) - plsc.load_gather(gc, [ridx, c2]))
        n = jnp.maximum(rx * rx + ry * ry + rz * rz, 1e-16)
        inv = _rsqrt_sc(n)
        x = rx * inv
        y = ry * inv
        z = rz * inv
        x2 = x * x
        y2 = y * y
        z2 = z * z
        xz = x * z
        cols = (x, y, z, x2, y2, z2, xz, x)
        for j in range(S_W):
            plsc.store_scatter(sbuf, [ridx, jnp.full((LANES,), j, jnp.int32)],
                               cols[j])


def _gather_kernel_body(pos16_hbm, f1_hbm, row_hbm, col_hbm,
                        s_out, f1c_out, *scr):
    rA = scr[0:3]
    rB = scr[3:6]
    cA = scr[6:9]
    cB = scr[9:12]
    gr = scr[12:15]
    gc = scr[15:18]
    gf = scr[18:21]
    sb = scr[21:24]
    rt, ct, grt, gct, gft, st = scr[24:30]
    semi = scr[30:33]
    semg = scr[33:36]
    semw = scr[36:39]

    wid = lax.axis_index("s") * NC + lax.axis_index("c")
    base = wid * E_PER_W

    def issue_idx(b, c):
        off = base + c * CH
        pltpu.async_copy(row_hbm.at[pl.ds(off, IDXV)], rA[b], semi[b])
        pltpu.async_copy(row_hbm.at[pl.ds(off + IDXV, IDXV)], rB[b], semi[b])
        pltpu.async_copy(col_hbm.at[pl.ds(off, IDXV)], cA[b], semi[b])
        pltpu.async_copy(col_hbm.at[pl.ds(off + IDXV, IDXV)], cB[b], semi[b])

    def wait_idx(b):
        pltpu.make_async_copy(row_hbm.at[pl.ds(0, IDXV)], rA[b], semi[b]).wait()
        pltpu.make_async_copy(row_hbm.at[pl.ds(0, IDXV)], rB[b], semi[b]).wait()
        pltpu.make_async_copy(col_hbm.at[pl.ds(0, IDXV)], cA[b], semi[b]).wait()
        pltpu.make_async_copy(col_hbm.at[pl.ds(0, IDXV)], cB[b], semi[b]).wait()

    def issue_gathers(b):
        pltpu.async_copy(pos16_hbm.at[rA[b]], gr[b].at[pl.ds(0, IDXV)], semg[b])
        pltpu.async_copy(pos16_hbm.at[rB[b]], gr[b].at[pl.ds(IDXV, IDXV)], semg[b])
        pltpu.async_copy(pos16_hbm.at[cA[b]], gc[b].at[pl.ds(0, IDXV)], semg[b])
        pltpu.async_copy(pos16_hbm.at[cB[b]], gc[b].at[pl.ds(IDXV, IDXV)], semg[b])
        pltpu.async_copy(f1_hbm.at[cA[b]], gf[b].at[pl.ds(0, IDXV)], semg[b])
        pltpu.async_copy(f1_hbm.at[cB[b]], gf[b].at[pl.ds(IDXV, IDXV)], semg[b])

    def wait_gathers(b):
        pltpu.make_async_copy(pos16_hbm.at[rA[b]], gr[b].at[pl.ds(0, IDXV)], semg[b]).wait()
        pltpu.make_async_copy(pos16_hbm.at[rB[b]], gr[b].at[pl.ds(IDXV, IDXV)], semg[b]).wait()
        pltpu.make_async_copy(pos16_hbm.at[cA[b]], gc[b].at[pl.ds(0, IDXV)], semg[b]).wait()
        pltpu.make_async_copy(pos16_hbm.at[cB[b]], gc[b].at[pl.ds(IDXV, IDXV)], semg[b]).wait()
        pltpu.make_async_copy(f1_hbm.at[cA[b]], gf[b].at[pl.ds(0, IDXV)], semg[b]).wait()
        pltpu.make_async_copy(f1_hbm.at[cB[b]], gf[b].at[pl.ds(IDXV, IDXV)], semg[b]).wait()

    def issue_writes(b, c):
        off = base + c * CH
        pltpu.async_copy(sb[b], s_out.at[pl.ds(off, CH)], semw[b])
        pltpu.async_copy(gf[b], f1c_out.at[pl.ds(off, CH)], semw[b])

    def wait_writes(b):
        pltpu.make_async_copy(sb[b], s_out.at[pl.ds(0, CH)], semw[b]).wait()
        pltpu.make_async_copy(gf[b], f1c_out.at[pl.ds(0, CH)], semw[b]).wait()

    def slot(c, b, *, wait_w=True, gather_next=True, idx_next=True,
             tail_work=False):
        nb = (b + 1) % 3
        b2 = (b + 2) % 3
        if wait_w:
            wait_writes(nb)
        if tail_work:
            pltpu.make_async_copy(row_hbm.at[pl.ds(0, TAIL)], rt, semi[0]).wait()
            pltpu.make_async_copy(col_hbm.at[pl.ds(0, TAIL)], ct, semi[0]).wait()
            pltpu.async_copy(pos16_hbm.at[rt], grt, semg[0])
            pltpu.async_copy(pos16_hbm.at[ct], gct, semg[0])
            pltpu.async_copy(f1_hbm.at[ct], gft, semg[0])
        if gather_next:
            wait_idx(nb)
            issue_gathers(nb)
        wait_gathers(b)
        if idx_next:
            issue_idx(b2, c + 2)
        _geom_chunk(CH, gr[b], gc[b], sb[b])
        issue_writes(b, c)

    issue_idx(0, 0)
    issue_idx(1, 1)
    wait_idx(0)
    issue_gathers(0)
    slot(0, 0, wait_w=False)
    slot(1, 1, wait_w=False)
    slot(2, 2)

    @pl.loop(1, 12)
    def _(i):
        c = 3 * i
        slot(c, 0)
        slot(c + 1, 1)
        slot(c + 2, 2)

    slot(36, 0)
    off_t = base + NFULL * CH
    pltpu.async_copy(row_hbm.at[pl.ds(off_t, TAIL)], rt, semi[0])
    pltpu.async_copy(col_hbm.at[pl.ds(off_t, TAIL)], ct, semi[0])
    slot(37, 1, idx_next=False)
    slot(38, 2, gather_next=False, idx_next=False, tail_work=True)
    pltpu.make_async_copy(pos16_hbm.at[rt], grt, semg[0]).wait()
    pltpu.make_async_copy(pos16_hbm.at[ct], gct, semg[0]).wait()
    pltpu.make_async_copy(f1_hbm.at[ct], gft, semg[0]).wait()
    _geom_chunk(TAIL, grt, gct, st)
    pltpu.async_copy(st, s_out.at[pl.ds(off_t, TAIL)], semw[0])
    pltpu.async_copy(gft, f1c_out.at[pl.ds(off_t, TAIL)], semw[0])
    wait_writes(1)
    wait_writes(2)
    pltpu.make_async_copy(st, s_out.at[pl.ds(0, TAIL)], semw[0]).wait()
    pltpu.make_async_copy(gft, f1c_out.at[pl.ds(0, TAIL)], semw[0]).wait()


def _sc_gather(pos16, f1, row, col):
    mesh = plsc.VectorSubcoreMesh(core_axis_name="c", subcore_axis_name="s",
                                  num_cores=NC, num_subcores=NS)
    idx3 = [pltpu.VMEM((IDXV,), jnp.int32)] * 12
    big3 = ([pltpu.VMEM((CH, POS_W), _F32)] * 6
            + [pltpu.VMEM((CH, HID), _F32)] * 3
            + [pltpu.VMEM((CH, S_W), _F32)] * 3)
    tails = [pltpu.VMEM((TAIL,), jnp.int32)] * 2 + [
        pltpu.VMEM((TAIL, POS_W), _F32),
        pltpu.VMEM((TAIL, POS_W), _F32),
        pltpu.VMEM((TAIL, HID), _F32),
        pltpu.VMEM((TAIL, S_W), _F32),
    ]
    fn = pl.kernel(
        _gather_kernel_body,
        out_type=(jax.ShapeDtypeStruct((N_EDGES, S_W), _F32),
                  jax.ShapeDtypeStruct((N_EDGES, HID), _F32)),
        mesh=mesh,
        scratch_types=idx3 + big3 + tails + [pltpu.SemaphoreType.DMA] * 9,
        compiler_params=pltpu.CompilerParams(use_tc_tiling_on_sc=False,
                                             needs_layout_passes=False),
    )
    return fn(pos16, f1, row, col)


def _mlp_body(shat_ref, f1hat_ref, mhat_ref, w2hat_ref, b2hat_ref, out_ref):
    u = (jnp.dot(shat_ref[...], mhat_ref[...], preferred_element_type=_F32)
         + f1hat_ref[...])
    h1 = _silu(u)
    h2 = _silu(jnp.dot(h1, w2hat_ref[...], preferred_element_type=_F32)
               + b2hat_ref[...])
    out_ref[...] = h2


def _mlp(shat, f1hat, Mhat, W2hat, b2hat):
    rows = N_EDGES // PACK
    blk = 2000
    grid = rows // blk
    return pl.pallas_call(
        _mlp_body,
        grid=(grid,),
        in_specs=[
            pl.BlockSpec((blk, PACK * S_W), lambda i: (i, 0)),
            pl.BlockSpec((blk, PACK * HID), lambda i: (i, 0)),
            pl.BlockSpec((PACK * S_W, PACK * HID), lambda i: (0, 0)),
            pl.BlockSpec((PACK * HID, PACK * HID), lambda i: (0, 0)),
            pl.BlockSpec((1, PACK * HID), lambda i: (0, 0)),
        ],
        out_specs=pl.BlockSpec((blk, PACK * HID), lambda i: (i, 0)),
        out_shape=jax.ShapeDtypeStruct((rows, PACK * HID), _F32),
    )(shat, f1hat, Mhat, W2hat, b2hat)


def _scatter_kernel_body(h2_hbm, row_hbm, zeros_h_hbm, zeros_d_hbm, degrows_hbm,
                         ph_out, pd_out, *scr):
    rA = scr[0:4]
    rB = scr[4:8]
    mb = scr[8:12]
    degbuf = scr[12]
    rt, mt = scr[13:15]
    acc_h, acc_d = scr[15:17]
    seml = scr[17:21]
    sema = scr[21:25]
    semt = scr[25]

    c_ax = lax.axis_index("c")
    s_ax = lax.axis_index("s")
    wid = s_ax * NC + c_ax
    stripe = pl.ds(s_ax * ROWS_PER_SUB, ROWS_PER_SUB)
    pltpu.sync_copy(zeros_h_hbm.at[stripe], acc_h.at[stripe])
    pltpu.sync_copy(zeros_d_hbm.at[stripe], acc_d.at[stripe])
    pltpu.sync_copy(degrows_hbm, degbuf)
    plsc.subcore_barrier()

    base = wid * E_PER_W

    def issue_loads(b, c):
        off = base + c * CH
        pltpu.async_copy(row_hbm.at[pl.ds(off, IDXV)], rA[b], seml[b])
        pltpu.async_copy(row_hbm.at[pl.ds(off + IDXV, IDXV)], rB[b], seml[b])
        pltpu.async_copy(h2_hbm.at[pl.ds(off, CH)], mb[b], seml[b])

    def wait_loads(b):
        pltpu.make_async_copy(row_hbm.at[pl.ds(0, IDXV)], rA[b], seml[b]).wait()
        pltpu.make_async_copy(row_hbm.at[pl.ds(0, IDXV)], rB[b], seml[b]).wait()
        pltpu.make_async_copy(h2_hbm.at[pl.ds(0, CH)], mb[b], seml[b]).wait()

    def issue_adds(b):
        pltpu.async_copy(mb[b].at[pl.ds(0, IDXV)], acc_h.at[rA[b]], sema[b],
                         add=True)
        pltpu.async_copy(mb[b].at[pl.ds(IDXV, IDXV)], acc_h.at[rB[b]], sema[b],
                         add=True)
        pltpu.async_copy(degbuf, acc_d.at[rA[b]], sema[b], add=True)
        pltpu.async_copy(degbuf, acc_d.at[rB[b]], sema[b], add=True)

    def wait_adds(b):
        pltpu.make_async_copy(mb[b].at[pl.ds(0, IDXV)], acc_h.at[rA[b]],
                              sema[b]).wait()
        pltpu.make_async_copy(mb[b].at[pl.ds(IDXV, IDXV)], acc_h.at[rB[b]],
                              sema[b]).wait()
        pltpu.make_async_copy(degbuf, acc_d.at[rA[b]], sema[b]).wait()
        pltpu.make_async_copy(degbuf, acc_d.at[rB[b]], sema[b]).wait()

    def slot(c, b, *, wait_a=True, loads_next=True):
        wait_loads(b)
        issue_adds(b)
        if wait_a:
            wait_adds((b + 2) % 4)
        if loads_next:
            issue_loads((b + 2) % 4, c + 2)

    issue_loads(0, 0)
    issue_loads(1, 1)
    slot(0, 0, wait_a=False)
    slot(1, 1, wait_a=False)

    @pl.loop(0, 8)
    def _(i):
        c = 2 + 4 * i
        slot(c, 2)
        slot(c + 1, 3)
        slot(c + 2, 0)
        slot(c + 3, 1)

    slot(34, 2)
    slot(35, 3)
    slot(36, 0)
    slot(37, 1, loads_next=False)
    slot(38, 2, loads_next=False)
    off_t = base + NFULL * CH
    pltpu.async_copy(row_hbm.at[pl.ds(off_t, TAIL)], rt, semt)
    pltpu.async_copy(h2_hbm.at[pl.ds(off_t, TAIL)], mt, semt)
    wait_adds(1)
    pltpu.make_async_copy(row_hbm.at[pl.ds(0, TAIL)], rt, semt).wait()
    pltpu.make_async_copy(h2_hbm.at[pl.ds(0, TAIL)], mt, semt).wait()
    pltpu.async_copy(mt, acc_h.at[rt], semt, add=True)
    pltpu.async_copy(degbuf.at[pl.ds(0, TAIL)], acc_d.at[rt], semt, add=True)
    wait_adds(2)
    pltpu.make_async_copy(mt, acc_h.at[rt], semt).wait()
    pltpu.make_async_copy(degbuf.at[pl.ds(0, TAIL)], acc_d.at[rt], semt).wait()

    plsc.subcore_barrier()
    pltpu.sync_copy(acc_h.at[stripe], ph_out.at[c_ax].at[stripe])
    pltpu.sync_copy(acc_d.at[stripe], pd_out.at[c_ax].at[stripe])


def _sc_scatter(h2, row, zeros_h, zeros_d, degrows):
    mesh = plsc.VectorSubcoreMesh(core_axis_name="c", subcore_axis_name="s",
                                  num_cores=NC, num_subcores=NS)
    fn = pl.kernel(
        _scatter_kernel_body,
        out_type=(jax.ShapeDtypeStruct((NC, N_NODES, HID), _F32),
                  jax.ShapeDtypeStruct((NC, N_NODES, DEG_W), _F32)),
        mesh=mesh,
        scratch_types=(
            [pltpu.VMEM((IDXV,), jnp.int32)] * 8
            + [pltpu.VMEM((CH, HID), _F32)] * 4
            + [pltpu.VMEM((IDXV, DEG_W), _F32)]
            + [pltpu.VMEM((TAIL,), jnp.int32), pltpu.VMEM((TAIL, HID), _F32)]
            + [pltpu.VMEM_SHARED((N_NODES, HID), _F32),
               pltpu.VMEM_SHARED((N_NODES, DEG_W), _F32)]
            + [pltpu.SemaphoreType.DMA] * 9
        ),
        compiler_params=pltpu.CompilerParams(use_tc_tiling_on_sc=False),
    )
    return fn(h2, row, zeros_h, zeros_d, degrows)


def _final_body(ph_ref, pd_ref, w3_ref, b3r_ref, out_ref):
    p = ph_ref[0] + ph_ref[1]
    deg = pd_ref[0, :, 0:1] + pd_ref[1, :, 0:1]
    w3s = w3_ref[:, 0:OUT_C]
    for k in range(1, N_SH):
        w3s = w3s + w3_ref[:, k * OUT_C:(k + 1) * OUT_C]
    b3s = jnp.sum(b3r_ref[...], axis=0, keepdims=True)
    out_ref[...] = (jnp.dot(p, w3s, preferred_element_type=_F32) + deg * b3s)


def _final(ph, pd, W3, b3r16):
    blk = 2000
    grid = N_NODES // blk
    return pl.pallas_call(
        _final_body,
        grid=(grid,),
        in_specs=[
            pl.BlockSpec((NC, blk, HID), lambda i: (0, i, 0)),
            pl.BlockSpec((NC, blk, DEG_W), lambda i: (0, i, 0)),
            pl.BlockSpec((HID, N_SH * OUT_C), lambda i: (0, 0)),
            pl.BlockSpec((16, OUT_C), lambda i: (0, 0)),
        ],
        out_specs=pl.BlockSpec((blk, OUT_C), lambda i: (i, 0)),
        out_shape=jax.ShapeDtypeStruct((N_NODES, OUT_C), _F32),
    )(ph, pd, W3, b3r16)


_SH_C0 = 0.5 / math.sqrt(math.pi)
_SH_C1 = 0.5 * math.sqrt(3.0 / (2.0 * math.pi))
_SH_C2 = 0.5 * math.sqrt(3.0 / math.pi)
_SH_C4 = 0.25 * math.sqrt(15.0 / (2.0 * math.pi))
_SH_C5 = 0.5 * math.sqrt(15.0 / (2.0 * math.pi))
_SH_C6 = 0.25 * math.sqrt(5.0 / math.pi)


def _blockdiag(m, k):
    r, c = m.shape
    out = jnp.zeros((k * r, k * c), m.dtype)
    for i in range(k):
        out = out.at[i * r:(i + 1) * r, i * c:(i + 1) * c].set(m)
    return out


def kernel(features, positions, edge_index, W1, b1, W2, b2, W3, b3):
    ei = edge_index.astype(jnp.int32)
    row = ei[0]
    col = ei[1]

    pos16 = jnp.pad(positions.astype(_F32), ((0, 0), (0, POS_W - 3)))
    W1a = W1[:N_SH]
    W1b = W1[N_SH:]
    M = jnp.stack([
        _SH_C1 * (W1a[3] - W1a[1]),
        jnp.zeros((HID,), _F32),
        _SH_C2 * W1a[2],
        _SH_C4 * (W1a[4] + W1a[8]) - _SH_C6 * W1a[6],
        -_SH_C4 * (W1a[4] + W1a[8]) - _SH_C6 * W1a[6],
        2.0 * _SH_C6 * W1a[6],
        _SH_C5 * (W1a[7] - W1a[5]),
        jnp.zeros((HID,), _F32),
    ])
    b1e = (b1 + _SH_C0 * W1a[0]).reshape(1, HID)
    Mhat = _blockdiag(M, PACK)
    W2hat = _blockdiag(W2, PACK)
    b2hat = jnp.tile(b2, (PACK,)).reshape(1, PACK * HID)
    b3r16 = jnp.pad(b3.reshape(N_SH, OUT_C), ((0, 16 - N_SH), (0, 0)))
    zeros_h = jnp.zeros((N_NODES, HID), _F32)
    zeros_d = jnp.zeros((N_NODES, DEG_W), _F32)
    degrows = jnp.zeros((IDXV, DEG_W), _F32).at[:, 0].set(1.0)

    f1 = _prep(features, W1b, b1e)
    s_mat, f1c = _sc_gather(pos16, f1, row, col)
    shat = s_mat.reshape(N_EDGES // PACK, PACK * S_W)
    f1hat = f1c.reshape(N_EDGES // PACK, PACK * HID)
    h2hat = _mlp(shat, f1hat, Mhat, W2hat, b2hat)
    h2 = h2hat.reshape(N_EDGES, HID)
    ph, pd = _sc_scatter(h2, row, zeros_h, zeros_d, degrows)
    return _final(ph, pd, W3, b3r16)


# speedup vs baseline: 24.7635x; 1.2141x over previous
import functools
import math

import jax
import jax.numpy as jnp
from jax import lax
from jax.experimental import pallas as pl
from jax.experimental.pallas import tpu as pltpu
from jax.experimental.pallas import tpu_sc as plsc

N_NODES = 10000
N_EDGES = 320000
IN_C = 128
OUT_C = 128
HID = 32
N_SH = 9

POS_W = 16
S_W = 8
PACK = 4
DEG_W = 32

NC = 2
NS = 16
NW = NC * NS
E_PER_W = N_EDGES // NW
IDXV = 128
CH = 256
NFULL = E_PER_W // CH
TAIL = E_PER_W - NFULL * CH
ROWS_PER_SUB = N_NODES // NS
LANES = 16

_F32 = jnp.float32


def _silu(x):
    return x * (1.0 / (1.0 + jnp.exp(-x)))


def _prep_body(feat4_ref, w1b4_ref, b1h_ref, f1_ref):
    f1_ref[...] = (jnp.dot(feat4_ref[...], w1b4_ref[...],
                           preferred_element_type=_F32) + b1h_ref[...])


def _prep(feat4, W1b4, b1hat):
    rows = N_NODES // 4
    return pl.pallas_call(
        _prep_body,
        grid=(1,),
        in_specs=[
            pl.BlockSpec((rows, 4 * IN_C), lambda i: (0, 0)),
            pl.BlockSpec((4 * IN_C, 4 * HID), lambda i: (0, 0)),
            pl.BlockSpec((1, 4 * HID), lambda i: (0, 0)),
        ],
        out_specs=pl.BlockSpec((rows, 4 * HID), lambda i: (0, 0)),
        out_shape=jax.ShapeDtypeStruct((rows, 4 * HID), _F32),
    )(feat4, W1b4, b1hat)


def _rsqrt_sc(n):
    i = plsc.bitcast(n, jnp.int32)
    i = jnp.full_like(i, 0x5F3759DF) - lax.shift_right_logical(i, 1)
    y = plsc.bitcast(i, _F32)
    y = y * (1.5 - 0.5 * n * y * y)
    y = y * (1.5 - 0.5 * n * y * y)
    return y


def _geom_chunk(k, gr, gc, sbuf):
    base_iota = lax.iota(jnp.int32, LANES)

    @pl.loop(0, k // LANES)
    def _(g):
        ridx = base_iota + g * LANES
        c0 = jnp.zeros((LANES,), jnp.int32)
        c1 = jnp.full((LANES,), 1, jnp.int32)
        c2 = jnp.full((LANES,), 2, jnp.int32)
        rx = (plsc.load_gather(gr, [ridx, c0]) - plsc.load_gather(gc, [ridx, c0]))
        ry = (plsc.load_gather(gr, [ridx, c1]) - plsc.load_gather(gc, [ridx, c1]))
        rz = (plsc.load_gather(gr, [ridx, c2]) - plsc.load_gather(gc, [ridx, c2]))
        n = jnp.maximum(rx * rx + ry * ry + rz * rz, 1e-16)
        inv = _rsqrt_sc(n)
        x = rx * inv
        y = ry * inv
        z = rz * inv
        x2 = x * x
        y2 = y * y
        z2 = z * z
        xz = x * z
        cols = (x, y, z, x2, y2, z2, xz, x)
        for j in range(S_W):
            plsc.store_scatter(sbuf, [ridx, jnp.full((LANES,), j, jnp.int32)],
                               cols[j])


def _gather_kernel_body(pos16_hbm, f1_hbm, ei_hbm,
                        s_out, f1c_out, *scr):
    rA = scr[0:3]
    rB = scr[3:6]
    cA = scr[6:9]
    cB = scr[9:12]
    gr = scr[12:15]
    gc = scr[15:18]
    gf = scr[18:21]
    sb = scr[21:24]
    rt, ct, grt, gct, gft, st = scr[24:30]
    semi = scr[30:33]
    semg = scr[33:36]
    semw = scr[36:39]

    wid = lax.axis_index("s") * NC + lax.axis_index("c")
    base = wid * E_PER_W

    def issue_idx(b, c):
        off = base + c * CH
        pltpu.async_copy(ei_hbm.at[0, pl.ds(off, IDXV)], rA[b], semi[b])
        pltpu.async_copy(ei_hbm.at[0, pl.ds(off + IDXV, IDXV)], rB[b], semi[b])
        pltpu.async_copy(ei_hbm.at[1, pl.ds(off, IDXV)], cA[b], semi[b])
        pltpu.async_copy(ei_hbm.at[1, pl.ds(off + IDXV, IDXV)], cB[b], semi[b])

    def wait_idx(b):
        pltpu.make_async_copy(ei_hbm.at[0, pl.ds(0, IDXV)], rA[b], semi[b]).wait()
        pltpu.make_async_copy(ei_hbm.at[0, pl.ds(0, IDXV)], rB[b], semi[b]).wait()
        pltpu.make_async_copy(ei_hbm.at[1, pl.ds(0, IDXV)], cA[b], semi[b]).wait()
        pltpu.make_async_copy(ei_hbm.at[1, pl.ds(0, IDXV)], cB[b], semi[b]).wait()

    def issue_gathers(b):
        pltpu.async_copy(pos16_hbm.at[rA[b]], gr[b].at[pl.ds(0, IDXV)], semg[b])
        pltpu.async_copy(pos16_hbm.at[rB[b]], gr[b].at[pl.ds(IDXV, IDXV)], semg[b])
        pltpu.async_copy(pos16_hbm.at[cA[b]], gc[b].at[pl.ds(0, IDXV)], semg[b])
        pltpu.async_copy(pos16_hbm.at[cB[b]], gc[b].at[pl.ds(IDXV, IDXV)], semg[b])
        pltpu.async_copy(f1_hbm.at[cA[b]], gf[b].at[pl.ds(0, IDXV)], semg[b])
        pltpu.async_copy(f1_hbm.at[cB[b]], gf[b].at[pl.ds(IDXV, IDXV)], semg[b])

    def wait_gathers(b):
        pltpu.make_async_copy(pos16_hbm.at[rA[b]], gr[b].at[pl.ds(0, IDXV)], semg[b]).wait()
        pltpu.make_async_copy(pos16_hbm.at[rB[b]], gr[b].at[pl.ds(IDXV, IDXV)], semg[b]).wait()
        pltpu.make_async_copy(pos16_hbm.at[cA[b]], gc[b].at[pl.ds(0, IDXV)], semg[b]).wait()
        pltpu.make_async_copy(pos16_hbm.at[cB[b]], gc[b].at[pl.ds(IDXV, IDXV)], semg[b]).wait()
        pltpu.make_async_copy(f1_hbm.at[cA[b]], gf[b].at[pl.ds(0, IDXV)], semg[b]).wait()
        pltpu.make_async_copy(f1_hbm.at[cB[b]], gf[b].at[pl.ds(IDXV, IDXV)], semg[b]).wait()

    def issue_writes(b, c):
        off = base + c * CH
        pltpu.async_copy(sb[b], s_out.at[pl.ds(off, CH)], semw[b])
        pltpu.async_copy(gf[b], f1c_out.at[pl.ds(off, CH)], semw[b])

    def wait_writes(b):
        pltpu.make_async_copy(sb[b], s_out.at[pl.ds(0, CH)], semw[b]).wait()
        pltpu.make_async_copy(gf[b], f1c_out.at[pl.ds(0, CH)], semw[b]).wait()

    def slot(c, b, *, wait_w=True, gather_next=True, idx_next=True,
             tail_work=False):
        nb = (b + 1) % 3
        b2 = (b + 2) % 3
        if wait_w:
            wait_writes(nb)
        if tail_work:
            pltpu.make_async_copy(ei_hbm.at[0, pl.ds(0, TAIL)], rt, semi[0]).wait()
            pltpu.make_async_copy(ei_hbm.at[1, pl.ds(0, TAIL)], ct, semi[0]).wait()
            pltpu.async_copy(pos16_hbm.at[rt], grt, semg[0])
            pltpu.async_copy(pos16_hbm.at[ct], gct, semg[0])
            pltpu.async_copy(f1_hbm.at[ct], gft, semg[0])
        if gather_next:
            wait_idx(nb)
            issue_gathers(nb)
        wait_gathers(b)
        if idx_next:
            issue_idx(b2, c + 2)
        _geom_chunk(CH, gr[b], gc[b], sb[b])
        issue_writes(b, c)

    issue_idx(0, 0)
    issue_idx(1, 1)
    wait_idx(0)
    issue_gathers(0)
    slot(0, 0, wait_w=False)
    slot(1, 1, wait_w=False)
    slot(2, 2)

    @pl.loop(1, 12)
    def _(i):
        c = 3 * i
        slot(c, 0)
        slot(c + 1, 1)
        slot(c + 2, 2)

    slot(36, 0)
    off_t = base + NFULL * CH
    pltpu.async_copy(ei_hbm.at[0, pl.ds(off_t, TAIL)], rt, semi[0])
    pltpu.async_copy(ei_hbm.at[1, pl.ds(off_t, TAIL)], ct, semi[0])
    slot(37, 1, idx_next=False)
    slot(38, 2, gather_next=False, idx_next=False, tail_work=True)
    pltpu.make_async_copy(pos16_hbm.at[rt], grt, semg[0]).wait()
    pltpu.make_async_copy(pos16_hbm.at[ct], gct, semg[0]).wait()
    pltpu.make_async_copy(f1_hbm.at[ct], gft, semg[0]).wait()
    _geom_chunk(TAIL, grt, gct, st)
    pltpu.async_copy(st, s_out.at[pl.ds(off_t, TAIL)], semw[0])
    pltpu.async_copy(gft, f1c_out.at[pl.ds(off_t, TAIL)], semw[0])
    wait_writes(1)
    wait_writes(2)
    pltpu.make_async_copy(st, s_out.at[pl.ds(0, TAIL)], semw[0]).wait()
    pltpu.make_async_copy(gft, f1c_out.at[pl.ds(0, TAIL)], semw[0]).wait()


def _sc_gather(pos16, f1, ei):
    mesh = plsc.VectorSubcoreMesh(core_axis_name="c", subcore_axis_name="s",
                                  num_cores=NC, num_subcores=NS)
    idx3 = [pltpu.VMEM((IDXV,), jnp.int32)] * 12
    big3 = ([pltpu.VMEM((CH, POS_W), _F32)] * 6
            + [pltpu.VMEM((CH, HID), _F32)] * 3
            + [pltpu.VMEM((CH, S_W), _F32)] * 3)
    tails = [pltpu.VMEM((TAIL,), jnp.int32)] * 2 + [
        pltpu.VMEM((TAIL, POS_W), _F32),
        pltpu.VMEM((TAIL, POS_W), _F32),
        pltpu.VMEM((TAIL, HID), _F32),
        pltpu.VMEM((TAIL, S_W), _F32),
    ]
    fn = pl.kernel(
        _gather_kernel_body,
        out_type=(jax.ShapeDtypeStruct((N_EDGES, S_W), _F32),
                  jax.ShapeDtypeStruct((N_EDGES, HID), _F32)),
        mesh=mesh,
        scratch_types=idx3 + big3 + tails + [pltpu.SemaphoreType.DMA] * 9,
        compiler_params=pltpu.CompilerParams(use_tc_tiling_on_sc=False,
                                             needs_layout_passes=False),
    )
    return fn(pos16, f1, ei)


def _mlp_body(s16_ref, f1hat_ref, mhat16_ref, w2hat_ref, b2hat_ref, out_ref):
    u16 = jnp.dot(s16_ref[...], mhat16_ref[...], preferred_element_type=_F32)
    u = u16.reshape(out_ref.shape) + f1hat_ref[...]
    h1 = _silu(u)
    h2 = _silu(jnp.dot(h1, w2hat_ref[...], preferred_element_type=_F32)
               + b2hat_ref[...])
    out_ref[...] = h2


def _mlp(s16, f1hat, Mhat16, W2hat, b2hat):
    rows = N_EDGES // PACK
    blk = 4000
    grid = rows // blk
    return pl.pallas_call(
        _mlp_body,
        grid=(grid,),
        in_specs=[
            pl.BlockSpec((blk // 4, 16 * S_W), lambda i: (i, 0)),
            pl.BlockSpec((blk, PACK * HID), lambda i: (i, 0)),
            pl.BlockSpec((16 * S_W, 16 * HID), lambda i: (0, 0)),
            pl.BlockSpec((PACK * HID, PACK * HID), lambda i: (0, 0)),
            pl.BlockSpec((1, PACK * HID), lambda i: (0, 0)),
        ],
        out_specs=pl.BlockSpec((blk, PACK * HID), lambda i: (i, 0)),
        out_shape=jax.ShapeDtypeStruct((rows, PACK * HID), _F32),
    )(s16, f1hat, Mhat16, W2hat, b2hat)


def _scatter_kernel_body(h2_hbm, ei_hbm, zeros_h_hbm, zeros_d_hbm, degrows_hbm,
                         ph_out, pd_out, *scr):
    rA = scr[0:4]
    rB = scr[4:8]
    mb = scr[8:12]
    degbuf = scr[12]
    rt, mt = scr[13:15]
    acc_h, acc_d = scr[15:17]
    seml = scr[17:21]
    sema = scr[21:25]
    semt = scr[25]

    c_ax = lax.axis_index("c")
    s_ax = lax.axis_index("s")
    wid = s_ax * NC + c_ax
    stripe = pl.ds(s_ax * ROWS_PER_SUB, ROWS_PER_SUB)
    pltpu.sync_copy(zeros_h_hbm.at[stripe], acc_h.at[stripe])
    pltpu.sync_copy(zeros_d_hbm.at[stripe], acc_d.at[stripe])
    pltpu.sync_copy(degrows_hbm, degbuf)
    plsc.subcore_barrier()

    base = wid * E_PER_W

    def issue_loads(b, c):
        off = base + c * CH
        pltpu.async_copy(ei_hbm.at[0, pl.ds(off, IDXV)], rA[b], seml[b])
        pltpu.async_copy(ei_hbm.at[0, pl.ds(off + IDXV, IDXV)], rB[b], seml[b])
        pltpu.async_copy(h2_hbm.at[pl.ds(off, CH)], mb[b], seml[b])

    def wait_loads(b):
        pltpu.make_async_copy(ei_hbm.at[0, pl.ds(0, IDXV)], rA[b], seml[b]).wait()
        pltpu.make_async_copy(ei_hbm.at[0, pl.ds(0, IDXV)], rB[b], seml[b]).wait()
        pltpu.make_async_copy(h2_hbm.at[pl.ds(0, CH)], mb[b], seml[b]).wait()

    def issue_adds(b):
        pltpu.async_copy(mb[b].at[pl.ds(0, IDXV)], acc_h.at[rA[b]], sema[b],
                         add=True)
        pltpu.async_copy(mb[b].at[pl.ds(IDXV, IDXV)], acc_h.at[rB[b]], sema[b],
                         add=True)
        pltpu.async_copy(degbuf, acc_d.at[rA[b]], sema[b], add=True)
        pltpu.async_copy(degbuf, acc_d.at[rB[b]], sema[b], add=True)

    def wait_adds(b):
        pltpu.make_async_copy(mb[b].at[pl.ds(0, IDXV)], acc_h.at[rA[b]],
                              sema[b]).wait()
        pltpu.make_async_copy(mb[b].at[pl.ds(IDXV, IDXV)], acc_h.at[rB[b]],
                              sema[b]).wait()
        pltpu.make_async_copy(degbuf, acc_d.at[rA[b]], sema[b]).wait()
        pltpu.make_async_copy(degbuf, acc_d.at[rB[b]], sema[b]).wait()

    def slot(c, b, *, wait_a=True, loads_next=True):
        wait_loads(b)
        issue_adds(b)
        if wait_a:
            wait_adds((b + 2) % 4)
        if loads_next:
            issue_loads((b + 2) % 4, c + 2)

    issue_loads(0, 0)
    issue_loads(1, 1)
    slot(0, 0, wait_a=False)
    slot(1, 1, wait_a=False)

    @pl.loop(0, 8)
    def _(i):
        c = 2 + 4 * i
        slot(c, 2)
        slot(c + 1, 3)
        slot(c + 2, 0)
        slot(c + 3, 1)

    slot(34, 2)
    slot(35, 3)
    slot(36, 0)
    slot(37, 1, loads_next=False)
    slot(38, 2, loads_next=False)
    off_t = base + NFULL * CH
    pltpu.async_copy(ei_hbm.at[0, pl.ds(off_t, TAIL)], rt, semt)
    pltpu.async_copy(h2_hbm.at[pl.ds(off_t, TAIL)], mt, semt)
    wait_adds(1)
    pltpu.make_async_copy(ei_hbm.at[0, pl.ds(0, TAIL)], rt, semt).wait()
    pltpu.make_async_copy(h2_hbm.at[pl.ds(0, TAIL)], mt, semt).wait()
    pltpu.async_copy(mt, acc_h.at[rt], semt, add=True)
    pltpu.async_copy(degbuf.at[pl.ds(0, TAIL)], acc_d.at[rt], semt, add=True)
    wait_adds(2)
    pltpu.make_async_copy(mt, acc_h.at[rt], semt).wait()
    pltpu.make_async_copy(degbuf.at[pl.ds(0, TAIL)], acc_d.at[rt], semt).wait()

    plsc.subcore_barrier()
    pltpu.sync_copy(acc_h.at[stripe], ph_out.at[c_ax].at[stripe])
    pltpu.sync_copy(acc_d.at[stripe], pd_out.at[c_ax].at[stripe])


def _sc_scatter(h2, ei, zeros_h, zeros_d, degrows):
    mesh = plsc.VectorSubcoreMesh(core_axis_name="c", subcore_axis_name="s",
                                  num_cores=NC, num_subcores=NS)
    fn = pl.kernel(
        _scatter_kernel_body,
        out_type=(jax.ShapeDtypeStruct((NC, N_NODES, HID), _F32),
                  jax.ShapeDtypeStruct((NC, N_NODES, DEG_W), _F32)),
        mesh=mesh,
        scratch_types=(
            [pltpu.VMEM((IDXV,), jnp.int32)] * 8
            + [pltpu.VMEM((CH, HID), _F32)] * 4
            + [pltpu.VMEM((IDXV, DEG_W), _F32)]
            + [pltpu.VMEM((TAIL,), jnp.int32), pltpu.VMEM((TAIL, HID), _F32)]
            + [pltpu.VMEM_SHARED((N_NODES, HID), _F32),
               pltpu.VMEM_SHARED((N_NODES, DEG_W), _F32)]
            + [pltpu.SemaphoreType.DMA] * 9
        ),
        compiler_params=pltpu.CompilerParams(use_tc_tiling_on_sc=False),
    )
    return fn(h2, ei, zeros_h, zeros_d, degrows)


def _final_body(ph_ref, pd_ref, w3_ref, b3r_ref, out_ref):
    p4 = ph_ref[0] + ph_ref[1]
    d4 = pd_ref[0] + pd_ref[1]
    w3s = w3_ref[:, 0:OUT_C]
    for k in range(1, N_SH):
        w3s = w3s + w3_ref[:, k * OUT_C:(k + 1) * OUT_C]
    b3s = jnp.sum(b3r_ref[...], axis=0, keepdims=True)
    for j in range(4):
        out_ref[:, j * OUT_C:(j + 1) * OUT_C] = (
            jnp.dot(p4[:, j * HID:(j + 1) * HID], w3s,
                    preferred_element_type=_F32)
            + d4[:, j * HID:j * HID + 1] * b3s)


def _final(ph4, pd4, W3, b3r16):
    rows = N_NODES // 4
    return pl.pallas_call(
        _final_body,
        grid=(1,),
        in_specs=[
            pl.BlockSpec((NC, rows, 4 * HID), lambda i: (0, 0, 0)),
            pl.BlockSpec((NC, rows, 4 * DEG_W), lambda i: (0, 0, 0)),
            pl.BlockSpec((HID, N_SH * OUT_C), lambda i: (0, 0)),
            pl.BlockSpec((16, OUT_C), lambda i: (0, 0)),
        ],
        out_specs=pl.BlockSpec((rows, 4 * OUT_C), lambda i: (0, 0)),
        out_shape=jax.ShapeDtypeStruct((rows, 4 * OUT_C), _F32),
    )(ph4, pd4, W3, b3r16)


_SH_C0 = 0.5 / math.sqrt(math.pi)
_SH_C1 = 0.5 * math.sqrt(3.0 / (2.0 * math.pi))
_SH_C2 = 0.5 * math.sqrt(3.0 / math.pi)
_SH_C4 = 0.25 * math.sqrt(15.0 / (2.0 * math.pi))
_SH_C5 = 0.5 * math.sqrt(15.0 / (2.0 * math.pi))
_SH_C6 = 0.25 * math.sqrt(5.0 / math.pi)


def _blockdiag(m, k):
    r, c = m.shape
    e = jnp.eye(k, dtype=m.dtype)
    return (e[:, None, :, None] * m[None, :, None, :]).reshape(k * r, k * c)


def kernel(features, positions, edge_index, W1, b1, W2, b2, W3, b3):
    ei = edge_index.astype(jnp.int32)

    W1a = W1[:N_SH]
    W1b = W1[N_SH:]
    M = jnp.stack([
        _SH_C1 * (W1a[3] - W1a[1]),
        jnp.zeros((HID,), _F32),
        _SH_C2 * W1a[2],
        _SH_C4 * (W1a[4] + W1a[8]) - _SH_C6 * W1a[6],
        -_SH_C4 * (W1a[4] + W1a[8]) - _SH_C6 * W1a[6],
        2.0 * _SH_C6 * W1a[6],
        _SH_C5 * (W1a[7] - W1a[5]),
        jnp.zeros((HID,), _F32),
    ])
    b1e = b1 + _SH_C0 * W1a[0]
    W1b4 = _blockdiag(W1b, 4)
    b1hat = jnp.tile(b1e, (4,)).reshape(1, 4 * HID)
    Mhat16 = _blockdiag(M, 16)
    W2hat = _blockdiag(W2, PACK)
    b2hat = jnp.tile(b2, (PACK,)).reshape(1, PACK * HID)
    b3r16 = jnp.pad(b3.reshape(N_SH, OUT_C), ((0, 16 - N_SH), (0, 0)))
    zeros_h = jnp.zeros((N_NODES, HID), _F32)
    zeros_d = jnp.zeros((N_NODES, DEG_W), _F32)
    degrows = jnp.zeros((IDXV, DEG_W), _F32).at[:, 0].set(1.0)

    feat4 = features.reshape(N_NODES // 4, 4 * IN_C)
    f1hat4 = _prep(feat4, W1b4, b1hat)
    f1 = f1hat4.reshape(N_NODES, HID)
    pos16 = jnp.pad(positions.astype(_F32), ((0, 0), (0, POS_W - 3)))
    s_mat, f1c = _sc_gather(pos16, f1, ei)
    s16 = s_mat.reshape(N_EDGES // 16, 16 * S_W)
    f1hat = f1c.reshape(N_EDGES // PACK, PACK * HID)
    h2hat = _mlp(s16, f1hat, Mhat16, W2hat, b2hat)
    h2 = h2hat.reshape(N_EDGES, HID)
    ph, pd = _sc_scatter(h2, ei, zeros_h, zeros_d, degrows)
    ph4 = ph.reshape(NC, N_NODES // 4, 4 * HID)
    pd4 = pd.reshape(NC, N_NODES // 4, 4 * DEG_W)
    out4 = _final(ph4, pd4, W3, b3r16)
    return out4.reshape(N_NODES, OUT_C)


# speedup vs baseline: 26.3870x; 1.0656x over previous
import functools
import math

import jax
import jax.numpy as jnp
from jax import lax
from jax.experimental import pallas as pl
from jax.experimental.pallas import tpu as pltpu
from jax.experimental.pallas import tpu_sc as plsc

N_NODES = 10000
N_EDGES = 320000
IN_C = 128
OUT_C = 128
HID = 32
N_SH = 9

POS_W = 16
S_W = 8
PACK = 4
DEG_W = 32

NC = 2
NS = 16
NW = NC * NS
E_PER_W = N_EDGES // NW
IDXV = 128
CH = 256
NFULL = E_PER_W // CH
TAIL = E_PER_W - NFULL * CH
ROWS_PER_SUB = N_NODES // NS
LANES = 16

_F32 = jnp.float32


def _silu(x):
    return (0.5 * x) * (1.0 + jnp.tanh(0.5 * x))


def _prep_body(feat4_ref, w1b4_ref, b1h_ref, f1_ref):
    f1_ref[...] = (jnp.dot(feat4_ref[...], w1b4_ref[...],
                           preferred_element_type=_F32) + b1h_ref[...])


def _prep(feat4, W1b4, b1hat):
    rows = N_NODES // 4
    return pl.pallas_call(
        _prep_body,
        grid=(1,),
        in_specs=[
            pl.BlockSpec((rows, 4 * IN_C), lambda i: (0, 0)),
            pl.BlockSpec((4 * IN_C, 4 * HID), lambda i: (0, 0)),
            pl.BlockSpec((1, 4 * HID), lambda i: (0, 0)),
        ],
        out_specs=pl.BlockSpec((rows, 4 * HID), lambda i: (0, 0)),
        out_shape=jax.ShapeDtypeStruct((rows, 4 * HID), _F32),
    )(feat4, W1b4, b1hat)


def _rsqrt_sc(n):
    i = plsc.bitcast(n, jnp.int32)
    i = jnp.full_like(i, 0x5F3759DF) - lax.shift_right_logical(i, 1)
    y = plsc.bitcast(i, _F32)
    y = y * (1.5 - 0.5 * n * y * y)
    y = y * (1.5 - 0.5 * n * y * y)
    return y


def _geom_chunk(k, gr, gc, sbuf):
    base_iota = lax.iota(jnp.int32, LANES)

    @pl.loop(0, k // LANES)
    def _(g):
        ridx = base_iota + g * LANES
        c0 = jnp.zeros((LANES,), jnp.int32)
        c1 = jnp.full((LANES,), 1, jnp.int32)
        c2 = jnp.full((LANES,), 2, jnp.int32)
        rx = (plsc.load_gather(gr, [ridx, c0]) - plsc.load_gather(gc, [ridx, c0]))
        ry = (plsc.load_gather(gr, [ridx, c1]) - plsc.load_gather(gc, [ridx, c1]))
        rz = (plsc.load_gather(gr, [ridx, c2]) - plsc.load_gather(gc, [ridx, c2]))
        n = jnp.maximum(rx * rx + ry * ry + rz * rz, 1e-16)
        inv = _rsqrt_sc(n)
        x = rx * inv
        y = ry * inv
        z = rz * inv
        x2 = x * x
        y2 = y * y
        z2 = z * z
        xz = x * z
        cols = (x, y, z, x2, y2, z2, xz, x)
        for j in range(S_W):
            plsc.store_scatter(sbuf, [ridx, jnp.full((LANES,), j, jnp.int32)],
                               cols[j])


def _gather_kernel_body(pos16_hbm, f1_hbm, ei_hbm,
                        s_out, f1c_out, *scr):
    rA = scr[0:3]
    rB = scr[3:6]
    cA = scr[6:9]
    cB = scr[9:12]
    gr = scr[12:15]
    gc = scr[15:18]
    gf = scr[18:21]
    sb = scr[21:24]
    rt, ct, grt, gct, gft, st = scr[24:30]
    semi = scr[30:33]
    semg = scr[33:36]
    semw = scr[36:39]

    wid = lax.axis_index("s") * NC + lax.axis_index("c")
    base = wid * E_PER_W

    def issue_idx(b, c):
        off = base + c * CH
        pltpu.async_copy(ei_hbm.at[0, pl.ds(off, IDXV)], rA[b], semi[b])
        pltpu.async_copy(ei_hbm.at[0, pl.ds(off + IDXV, IDXV)], rB[b], semi[b])
        pltpu.async_copy(ei_hbm.at[1, pl.ds(off, IDXV)], cA[b], semi[b])
        pltpu.async_copy(ei_hbm.at[1, pl.ds(off + IDXV, IDXV)], cB[b], semi[b])

    def wait_idx(b):
        pltpu.make_async_copy(ei_hbm.at[0, pl.ds(0, IDXV)], rA[b], semi[b]).wait()
        pltpu.make_async_copy(ei_hbm.at[0, pl.ds(0, IDXV)], rB[b], semi[b]).wait()
        pltpu.make_async_copy(ei_hbm.at[1, pl.ds(0, IDXV)], cA[b], semi[b]).wait()
        pltpu.make_async_copy(ei_hbm.at[1, pl.ds(0, IDXV)], cB[b], semi[b]).wait()

    def issue_gathers(b):
        pltpu.async_copy(pos16_hbm.at[rA[b]], gr[b].at[pl.ds(0, IDXV)], semg[b])
        pltpu.async_copy(pos16_hbm.at[rB[b]], gr[b].at[pl.ds(IDXV, IDXV)], semg[b])
        pltpu.async_copy(pos16_hbm.at[cA[b]], gc[b].at[pl.ds(0, IDXV)], semg[b])
        pltpu.async_copy(pos16_hbm.at[cB[b]], gc[b].at[pl.ds(IDXV, IDXV)], semg[b])
        pltpu.async_copy(f1_hbm.at[cA[b]], gf[b].at[pl.ds(0, IDXV)], semg[b])
        pltpu.async_copy(f1_hbm.at[cB[b]], gf[b].at[pl.ds(IDXV, IDXV)], semg[b])

    def wait_gathers(b):
        pltpu.make_async_copy(pos16_hbm.at[rA[b]], gr[b].at[pl.ds(0, IDXV)], semg[b]).wait()
        pltpu.make_async_copy(pos16_hbm.at[rB[b]], gr[b].at[pl.ds(IDXV, IDXV)], semg[b]).wait()
        pltpu.make_async_copy(pos16_hbm.at[cA[b]], gc[b].at[pl.ds(0, IDXV)], semg[b]).wait()
        pltpu.make_async_copy(pos16_hbm.at[cB[b]], gc[b].at[pl.ds(IDXV, IDXV)], semg[b]).wait()
        pltpu.make_async_copy(f1_hbm.at[cA[b]], gf[b].at[pl.ds(0, IDXV)], semg[b]).wait()
        pltpu.make_async_copy(f1_hbm.at[cB[b]], gf[b].at[pl.ds(IDXV, IDXV)], semg[b]).wait()

    def issue_writes(b, c):
        off = base + c * CH
        pltpu.async_copy(sb[b], s_out.at[pl.ds(off, CH)], semw[b])
        pltpu.async_copy(gf[b], f1c_out.at[pl.ds(off, CH)], semw[b])

    def wait_writes(b):
        pltpu.make_async_copy(sb[b], s_out.at[pl.ds(0, CH)], semw[b]).wait()
        pltpu.make_async_copy(gf[b], f1c_out.at[pl.ds(0, CH)], semw[b]).wait()

    def slot(c, b, *, wait_w=True, gather_next=True, idx_next=True,
             tail_work=False):
        nb = (b + 1) % 3
        b2 = (b + 2) % 3
        if wait_w:
            wait_writes(nb)
        if tail_work:
            pltpu.make_async_copy(ei_hbm.at[0, pl.ds(0, TAIL)], rt, semi[0]).wait()
            pltpu.make_async_copy(ei_hbm.at[1, pl.ds(0, TAIL)], ct, semi[0]).wait()
            pltpu.async_copy(pos16_hbm.at[rt], grt, semg[0])
            pltpu.async_copy(pos16_hbm.at[ct], gct, semg[0])
            pltpu.async_copy(f1_hbm.at[ct], gft, semg[0])
        if gather_next:
            wait_idx(nb)
            issue_gathers(nb)
        wait_gathers(b)
        if idx_next:
            issue_idx(b2, c + 2)
        _geom_chunk(CH, gr[b], gc[b], sb[b])
        issue_writes(b, c)

    issue_idx(0, 0)
    issue_idx(1, 1)
    wait_idx(0)
    issue_gathers(0)
    slot(0, 0, wait_w=False)
    slot(1, 1, wait_w=False)
    slot(2, 2)

    @pl.loop(1, 12)
    def _(i):
        c = 3 * i
        slot(c, 0)
        slot(c + 1, 1)
        slot(c + 2, 2)

    slot(36, 0)
    off_t = base + NFULL * CH
    pltpu.async_copy(ei_hbm.at[0, pl.ds(off_t, TAIL)], rt, semi[0])
    pltpu.async_copy(ei_hbm.at[1, pl.ds(off_t, TAIL)], ct, semi[0])
    slot(37, 1, idx_next=False)
    slot(38, 2, gather_next=False, idx_next=False, tail_work=True)
    pltpu.make_async_copy(pos16_hbm.at[rt], grt, semg[0]).wait()
    pltpu.make_async_copy(pos16_hbm.at[ct], gct, semg[0]).wait()
    pltpu.make_async_copy(f1_hbm.at[ct], gft, semg[0]).wait()
    _geom_chunk(TAIL, grt, gct, st)
    pltpu.async_copy(st, s_out.at[pl.ds(off_t, TAIL)], semw[0])
    pltpu.async_copy(gft, f1c_out.at[pl.ds(off_t, TAIL)], semw[0])
    wait_writes(1)
    wait_writes(2)
    pltpu.make_async_copy(st, s_out.at[pl.ds(0, TAIL)], semw[0]).wait()
    pltpu.make_async_copy(gft, f1c_out.at[pl.ds(0, TAIL)], semw[0]).wait()


def _sc_gather(pos16, f1, ei):
    mesh = plsc.VectorSubcoreMesh(core_axis_name="c", subcore_axis_name="s",
                                  num_cores=NC, num_subcores=NS)
    idx3 = [pltpu.VMEM((IDXV,), jnp.int32)] * 12
    big3 = ([pltpu.VMEM((CH, POS_W), _F32)] * 6
            + [pltpu.VMEM((CH, HID), _F32)] * 3
            + [pltpu.VMEM((CH, S_W), _F32)] * 3)
    tails = [pltpu.VMEM((TAIL,), jnp.int32)] * 2 + [
        pltpu.VMEM((TAIL, POS_W), _F32),
        pltpu.VMEM((TAIL, POS_W), _F32),
        pltpu.VMEM((TAIL, HID), _F32),
        pltpu.VMEM((TAIL, S_W), _F32),
    ]
    fn = pl.kernel(
        _gather_kernel_body,
        out_type=(jax.ShapeDtypeStruct((N_EDGES, S_W), _F32),
                  jax.ShapeDtypeStruct((N_EDGES, HID), _F32)),
        mesh=mesh,
        scratch_types=idx3 + big3 + tails + [pltpu.SemaphoreType.DMA] * 9,
        compiler_params=pltpu.CompilerParams(use_tc_tiling_on_sc=False,
                                             needs_layout_passes=False),
    )
    return fn(pos16, f1, ei)


def _mlp_body(s16_ref, f1hat_ref, mhat16_ref, w2hat_ref, b2hat_ref, out_ref):
    u16 = jnp.dot(s16_ref[...], mhat16_ref[...], preferred_element_type=_F32)
    u = u16.reshape(out_ref.shape) + f1hat_ref[...]
    h1 = _silu(u)
    h2 = _silu(jnp.dot(h1, w2hat_ref[...], preferred_element_type=_F32)
               + b2hat_ref[...])
    out_ref[...] = h2


def _mlp(s16, f1hat, Mhat16, W2hat, b2hat):
    rows = N_EDGES // PACK
    blk = 4000
    grid = rows // blk
    return pl.pallas_call(
        _mlp_body,
        grid=(grid,),
        in_specs=[
            pl.BlockSpec((blk // 4, 16 * S_W), lambda i: (i, 0)),
            pl.BlockSpec((blk, PACK * HID), lambda i: (i, 0)),
            pl.BlockSpec((16 * S_W, 16 * HID), lambda i: (0, 0)),
            pl.BlockSpec((PACK * HID, PACK * HID), lambda i: (0, 0)),
            pl.BlockSpec((1, PACK * HID), lambda i: (0, 0)),
        ],
        out_specs=pl.BlockSpec((blk, PACK * HID), lambda i: (i, 0)),
        out_shape=jax.ShapeDtypeStruct((rows, PACK * HID), _F32),
    )(s16, f1hat, Mhat16, W2hat, b2hat)


def _scatter_kernel_body(h2_hbm, ei_hbm, zeros_h_hbm, zeros_d_hbm, degrows_hbm,
                         ph_out, pd_out, *scr):
    rA = scr[0:4]
    rB = scr[4:8]
    mb = scr[8:12]
    degbuf = scr[12]
    rt, mt = scr[13:15]
    acc_h, acc_d = scr[15:17]
    seml = scr[17:21]
    sema = scr[21:25]
    semt = scr[25]

    c_ax = lax.axis_index("c")
    s_ax = lax.axis_index("s")
    wid = s_ax * NC + c_ax
    stripe = pl.ds(s_ax * ROWS_PER_SUB, ROWS_PER_SUB)
    pltpu.sync_copy(zeros_h_hbm.at[stripe], acc_h.at[stripe])
    pltpu.sync_copy(zeros_d_hbm.at[stripe], acc_d.at[stripe])
    pltpu.sync_copy(degrows_hbm, degbuf)
    plsc.subcore_barrier()

    base = wid * E_PER_W

    def issue_loads(b, c):
        off = base + c * CH
        pltpu.async_copy(ei_hbm.at[0, pl.ds(off, IDXV)], rA[b], seml[b])
        pltpu.async_copy(ei_hbm.at[0, pl.ds(off + IDXV, IDXV)], rB[b], seml[b])
        pltpu.async_copy(h2_hbm.at[pl.ds(off, CH)], mb[b], seml[b])

    def wait_loads(b):
        pltpu.make_async_copy(ei_hbm.at[0, pl.ds(0, IDXV)], rA[b], seml[b]).wait()
        pltpu.make_async_copy(ei_hbm.at[0, pl.ds(0, IDXV)], rB[b], seml[b]).wait()
        pltpu.make_async_copy(h2_hbm.at[pl.ds(0, CH)], mb[b], seml[b]).wait()

    def issue_adds(b):
        pltpu.async_copy(mb[b].at[pl.ds(0, IDXV)], acc_h.at[rA[b]], sema[b],
                         add=True)
        pltpu.async_copy(mb[b].at[pl.ds(IDXV, IDXV)], acc_h.at[rB[b]], sema[b],
                         add=True)
        pltpu.async_copy(degbuf, acc_d.at[rA[b]], sema[b], add=True)
        pltpu.async_copy(degbuf, acc_d.at[rB[b]], sema[b], add=True)

    def wait_adds(b):
        pltpu.make_async_copy(mb[b].at[pl.ds(0, IDXV)], acc_h.at[rA[b]],
                              sema[b]).wait()
        pltpu.make_async_copy(mb[b].at[pl.ds(IDXV, IDXV)], acc_h.at[rB[b]],
                              sema[b]).wait()
        pltpu.make_async_copy(degbuf, acc_d.at[rA[b]], sema[b]).wait()
        pltpu.make_async_copy(degbuf, acc_d.at[rB[b]], sema[b]).wait()

    def slot(c, b, *, wait_a=True, loads_next=True):
        wait_loads(b)
        issue_adds(b)
        if wait_a:
            wait_adds((b + 2) % 4)
        if loads_next:
            issue_loads((b + 2) % 4, c + 2)

    issue_loads(0, 0)
    issue_loads(1, 1)
    slot(0, 0, wait_a=False)
    slot(1, 1, wait_a=False)

    @pl.loop(0, 8)
    def _(i):
        c = 2 + 4 * i
        slot(c, 2)
        slot(c + 1, 3)
        slot(c + 2, 0)
        slot(c + 3, 1)

    slot(34, 2)
    slot(35, 3)
    slot(36, 0)
    slot(37, 1, loads_next=False)
    slot(38, 2, loads_next=False)
    off_t = base + NFULL * CH
    pltpu.async_copy(ei_hbm.at[0, pl.ds(off_t, TAIL)], rt, semt)
    pltpu.async_copy(h2_hbm.at[pl.ds(off_t, TAIL)], mt, semt)
    wait_adds(1)
    pltpu.make_async_copy(ei_hbm.at[0, pl.ds(0, TAIL)], rt, semt).wait()
    pltpu.make_async_copy(h2_hbm.at[pl.ds(0, TAIL)], mt, semt).wait()
    pltpu.async_copy(mt, acc_h.at[rt], semt, add=True)
    pltpu.async_copy(degbuf.at[pl.ds(0, TAIL)], acc_d.at[rt], semt, add=True)
    wait_adds(2)
    pltpu.make_async_copy(mt, acc_h.at[rt], semt).wait()
    pltpu.make_async_copy(degbuf.at[pl.ds(0, TAIL)], acc_d.at[rt], semt).wait()

    plsc.subcore_barrier()
    pltpu.sync_copy(acc_h.at[stripe], ph_out.at[c_ax].at[stripe])
    pltpu.sync_copy(acc_d.at[stripe], pd_out.at[c_ax].at[stripe])


def _sc_scatter(h2, ei, zeros_h, zeros_d, degrows):
    mesh = plsc.VectorSubcoreMesh(core_axis_name="c", subcore_axis_name="s",
                                  num_cores=NC, num_subcores=NS)
    fn = pl.kernel(
        _scatter_kernel_body,
        out_type=(jax.ShapeDtypeStruct((NC, N_NODES, HID), _F32),
                  jax.ShapeDtypeStruct((NC, N_NODES, DEG_W), _F32)),
        mesh=mesh,
        scratch_types=(
            [pltpu.VMEM((IDXV,), jnp.int32)] * 8
            + [pltpu.VMEM((CH, HID), _F32)] * 4
            + [pltpu.VMEM((IDXV, DEG_W), _F32)]
            + [pltpu.VMEM((TAIL,), jnp.int32), pltpu.VMEM((TAIL, HID), _F32)]
            + [pltpu.VMEM_SHARED((N_NODES, HID), _F32),
               pltpu.VMEM_SHARED((N_NODES, DEG_W), _F32)]
            + [pltpu.SemaphoreType.DMA] * 9
        ),
        compiler_params=pltpu.CompilerParams(use_tc_tiling_on_sc=False),
    )
    return fn(h2, ei, zeros_h, zeros_d, degrows)


def _final_body(ph_ref, pd_ref, w3_ref, b3r_ref, out_ref):
    rows = N_NODES // 4
    p4 = ph_ref[0:rows] + ph_ref[rows:2 * rows]
    d4 = pd_ref[0:rows] + pd_ref[rows:2 * rows]
    w3s = w3_ref[:, 0:OUT_C]
    for k in range(1, N_SH):
        w3s = w3s + w3_ref[:, k * OUT_C:(k + 1) * OUT_C]
    b3s = jnp.sum(b3r_ref[...], axis=0, keepdims=True)
    cols = []
    for j in range(4):
        cols.append(jnp.dot(p4[:, j * HID:(j + 1) * HID], w3s,
                            preferred_element_type=_F32)
                    + d4[:, j * HID:j * HID + 1] * b3s)
    out4 = jnp.concatenate(cols, axis=1)
    out_ref[...] = out4.reshape(N_NODES, OUT_C)


def _final(ph2, pd2, W3, b3r16):
    rows = N_NODES // 4
    return pl.pallas_call(
        _final_body,
        grid=(1,),
        in_specs=[
            pl.BlockSpec((NC * rows, 4 * HID), lambda i: (0, 0)),
            pl.BlockSpec((NC * rows, 4 * DEG_W), lambda i: (0, 0)),
            pl.BlockSpec((HID, N_SH * OUT_C), lambda i: (0, 0)),
            pl.BlockSpec((16, OUT_C), lambda i: (0, 0)),
        ],
        out_specs=pl.BlockSpec((N_NODES, OUT_C), lambda i: (0, 0)),
        out_shape=jax.ShapeDtypeStruct((N_NODES, OUT_C), _F32),
    )(ph2, pd2, W3, b3r16)


_SH_C0 = 0.5 / math.sqrt(math.pi)
_SH_C1 = 0.5 * math.sqrt(3.0 / (2.0 * math.pi))
_SH_C2 = 0.5 * math.sqrt(3.0 / math.pi)
_SH_C4 = 0.25 * math.sqrt(15.0 / (2.0 * math.pi))
_SH_C5 = 0.5 * math.sqrt(15.0 / (2.0 * math.pi))
_SH_C6 = 0.25 * math.sqrt(5.0 / math.pi)


def _blockdiag(m, k):
    r, c = m.shape
    e = jnp.eye(k, dtype=m.dtype)
    return (e[:, None, :, None] * m[None, :, None, :]).reshape(k * r, k * c)


def kernel(features, positions, edge_index, W1, b1, W2, b2, W3, b3):
    ei = edge_index.astype(jnp.int32)

    W1a = W1[:N_SH]
    W1b = W1[N_SH:]
    M = jnp.stack([
        _SH_C1 * (W1a[3] - W1a[1]),
        jnp.zeros((HID,), _F32),
        _SH_C2 * W1a[2],
        _SH_C4 * (W1a[4] + W1a[8]) - _SH_C6 * W1a[6],
        -_SH_C4 * (W1a[4] + W1a[8]) - _SH_C6 * W1a[6],
        2.0 * _SH_C6 * W1a[6],
        _SH_C5 * (W1a[7] - W1a[5]),
        jnp.zeros((HID,), _F32),
    ])
    b1e = b1 + _SH_C0 * W1a[0]
    W1b4 = _blockdiag(W1b, 4)
    b1hat = jnp.tile(b1e, (4,)).reshape(1, 4 * HID)
    Mhat16 = _blockdiag(M, 16)
    W2hat = _blockdiag(W2, PACK)
    b2hat = jnp.tile(b2, (PACK,)).reshape(1, PACK * HID)
    b3r16 = jnp.pad(b3.reshape(N_SH, OUT_C), ((0, 16 - N_SH), (0, 0)))
    zeros_h = jnp.zeros((N_NODES, HID), _F32)
    zeros_d = jnp.zeros((N_NODES, DEG_W), _F32)
    degrows = jnp.zeros((IDXV, DEG_W), _F32).at[:, 0].set(1.0)

    feat4 = features.reshape(N_NODES // 4, 4 * IN_C)
    f1hat4 = _prep(feat4, W1b4, b1hat)
    f1 = f1hat4.reshape(N_NODES, HID)
    pos16 = jnp.pad(positions.astype(_F32), ((0, 0), (0, POS_W - 3)))
    s_mat, f1c = _sc_gather(pos16, f1, ei)
    s16 = s_mat.reshape(N_EDGES // 16, 16 * S_W)
    f1hat = f1c.reshape(N_EDGES // PACK, PACK * HID)
    h2hat = _mlp(s16, f1hat, Mhat16, W2hat, b2hat)
    h2 = h2hat.reshape(N_EDGES, HID)
    ph, pd = _sc_scatter(h2, ei, zeros_h, zeros_d, degrows)
    ph2 = ph.reshape(NC * N_NODES // 4, 4 * HID)
    pd2 = pd.reshape(NC * N_NODES // 4, 4 * DEG_W)
    return _final(ph2, pd2, W3, b3r16)


# speedup vs baseline: 27.0806x; 1.0263x over previous
import functools
import math

import jax
import jax.numpy as jnp
from jax import lax
from jax.experimental import pallas as pl
from jax.experimental.pallas import tpu as pltpu
from jax.experimental.pallas import tpu_sc as plsc

N_NODES = 10000
N_EDGES = 320000
IN_C = 128
OUT_C = 128
HID = 32
N_SH = 9

POS_W = 16
S_W = 8
PACK = 4
DEG_W = 32

NC = 2
NS = 16
NW = NC * NS
E_PER_W = N_EDGES // NW
IDXV = 128
CH = 256
NFULL = E_PER_W // CH
TAIL = E_PER_W - NFULL * CH
ROWS_PER_SUB = N_NODES // NS
LANES = 16

_F32 = jnp.float32


def _silu(x):
    return (0.5 * x) * (1.0 + jnp.tanh(0.5 * x))


def _prep_body(feat4_ref, w1b4_ref, b1h_ref, f1_ref):
    f1_ref[...] = (jnp.dot(feat4_ref[...], w1b4_ref[...],
                           preferred_element_type=_F32) + b1h_ref[...])


def _prep(feat4, W1b4, b1hat):
    rows = N_NODES // 4
    return pl.pallas_call(
        _prep_body,
        grid=(1,),
        in_specs=[
            pl.BlockSpec((rows, 4 * IN_C), lambda i: (0, 0)),
            pl.BlockSpec((4 * IN_C, 4 * HID), lambda i: (0, 0)),
            pl.BlockSpec((1, 4 * HID), lambda i: (0, 0)),
        ],
        out_specs=pl.BlockSpec((rows, 4 * HID), lambda i: (0, 0)),
        out_shape=jax.ShapeDtypeStruct((rows, 4 * HID), _F32),
    )(feat4, W1b4, b1hat)


def _rsqrt_sc(n):
    i = plsc.bitcast(n, jnp.int32)
    i = jnp.full_like(i, 0x5F3759DF) - lax.shift_right_logical(i, 1)
    y = plsc.bitcast(i, _F32)
    y = y * (1.5 - 0.5 * n * y * y)
    y = y * (1.5 - 0.5 * n * y * y)
    return y


def _geom_chunk(k, gr, gc, sbuf):
    base_iota = lax.iota(jnp.int32, LANES)

    @pl.loop(0, k // LANES)
    def _(g):
        ridx = base_iota + g * LANES
        c0 = jnp.zeros((LANES,), jnp.int32)
        c1 = jnp.full((LANES,), 1, jnp.int32)
        c2 = jnp.full((LANES,), 2, jnp.int32)
        rx = (plsc.load_gather(gr, [ridx, c0]) - plsc.load_gather(gc, [ridx, c0]))
        ry = (plsc.load_gather(gr, [ridx, c1]) - plsc.load_gather(gc, [ridx, c1]))
        rz = (plsc.load_gather(gr, [ridx, c2]) - plsc.load_gather(gc, [ridx, c2]))
        n = jnp.maximum(rx * rx + ry * ry + rz * rz, 1e-16)
        inv = _rsqrt_sc(n)
        x = rx * inv
        y = ry * inv
        z = rz * inv
        x2 = x * x
        y2 = y * y
        z2 = z * z
        xz = x * z
        cols = (x, y, z, x2, y2, z2, xz, x)
        for j in range(S_W):
            plsc.store_scatter(sbuf, [ridx, jnp.full((LANES,), j, jnp.int32)],
                               cols[j])


def _gather_kernel_body(pos16_hbm, f1_hbm, ei_hbm,
                        s_out, f1c_out, *scr):
    rA = scr[0:3]
    rB = scr[3:6]
    cA = scr[6:9]
    cB = scr[9:12]
    gr = scr[12:15]
    gc = scr[15:18]
    gf = scr[18:21]
    sb = scr[21:24]
    rt, ct, grt, gct, gft, st = scr[24:30]
    semi = scr[30:33]
    semg = scr[33:36]
    semw = scr[36:39]

    wid = lax.axis_index("s") * NC + lax.axis_index("c")
    base = wid * E_PER_W

    def issue_idx(b, c):
        off = base + c * CH
        pltpu.async_copy(ei_hbm.at[0, pl.ds(off, IDXV)], rA[b], semi[b])
        pltpu.async_copy(ei_hbm.at[0, pl.ds(off + IDXV, IDXV)], rB[b], semi[b])
        pltpu.async_copy(ei_hbm.at[1, pl.ds(off, IDXV)], cA[b], semi[b])
        pltpu.async_copy(ei_hbm.at[1, pl.ds(off + IDXV, IDXV)], cB[b], semi[b])

    def wait_idx(b):
        pltpu.make_async_copy(ei_hbm.at[0, pl.ds(0, IDXV)], rA[b], semi[b]).wait()
        pltpu.make_async_copy(ei_hbm.at[0, pl.ds(0, IDXV)], rB[b], semi[b]).wait()
        pltpu.make_async_copy(ei_hbm.at[1, pl.ds(0, IDXV)], cA[b], semi[b]).wait()
        pltpu.make_async_copy(ei_hbm.at[1, pl.ds(0, IDXV)], cB[b], semi[b]).wait()

    def issue_gathers(b):
        pltpu.async_copy(pos16_hbm.at[rA[b]], gr[b].at[pl.ds(0, IDXV)], semg[b])
        pltpu.async_copy(pos16_hbm.at[rB[b]], gr[b].at[pl.ds(IDXV, IDXV)], semg[b])
        pltpu.async_copy(pos16_hbm.at[cA[b]], gc[b].at[pl.ds(0, IDXV)], semg[b])
        pltpu.async_copy(pos16_hbm.at[cB[b]], gc[b].at[pl.ds(IDXV, IDXV)], semg[b])
        pltpu.async_copy(f1_hbm.at[cA[b]], gf[b].at[pl.ds(0, IDXV)], semg[b])
        pltpu.async_copy(f1_hbm.at[cB[b]], gf[b].at[pl.ds(IDXV, IDXV)], semg[b])

    def wait_gathers(b):
        pltpu.make_async_copy(pos16_hbm.at[rA[b]], gr[b].at[pl.ds(0, IDXV)], semg[b]).wait()
        pltpu.make_async_copy(pos16_hbm.at[rB[b]], gr[b].at[pl.ds(IDXV, IDXV)], semg[b]).wait()
        pltpu.make_async_copy(pos16_hbm.at[cA[b]], gc[b].at[pl.ds(0, IDXV)], semg[b]).wait()
        pltpu.make_async_copy(pos16_hbm.at[cB[b]], gc[b].at[pl.ds(IDXV, IDXV)], semg[b]).wait()
        pltpu.make_async_copy(f1_hbm.at[cA[b]], gf[b].at[pl.ds(0, IDXV)], semg[b]).wait()
        pltpu.make_async_copy(f1_hbm.at[cB[b]], gf[b].at[pl.ds(IDXV, IDXV)], semg[b]).wait()

    def issue_writes(b, c):
        off = base + c * CH
        pltpu.async_copy(sb[b], s_out.at[pl.ds(off, CH)], semw[b])
        pltpu.async_copy(gf[b], f1c_out.at[pl.ds(off, CH)], semw[b])

    def wait_writes(b):
        pltpu.make_async_copy(sb[b], s_out.at[pl.ds(0, CH)], semw[b]).wait()
        pltpu.make_async_copy(gf[b], f1c_out.at[pl.ds(0, CH)], semw[b]).wait()

    def slot(c, b, *, wait_w=True, gather_next=True, idx_next=True,
             tail_work=False):
        nb = (b + 1) % 3
        b2 = (b + 2) % 3
        if wait_w:
            wait_writes(nb)
        if tail_work:
            pltpu.make_async_copy(ei_hbm.at[0, pl.ds(0, TAIL)], rt, semi[0]).wait()
            pltpu.make_async_copy(ei_hbm.at[1, pl.ds(0, TAIL)], ct, semi[0]).wait()
            pltpu.async_copy(pos16_hbm.at[rt], grt, semg[0])
            pltpu.async_copy(pos16_hbm.at[ct], gct, semg[0])
            pltpu.async_copy(f1_hbm.at[ct], gft, semg[0])
        if gather_next:
            wait_idx(nb)
            issue_gathers(nb)
        wait_gathers(b)
        if idx_next:
            issue_idx(b2, c + 2)
        _geom_chunk(CH, gr[b], gc[b], sb[b])
        issue_writes(b, c)

    issue_idx(0, 0)
    issue_idx(1, 1)
    wait_idx(0)
    issue_gathers(0)
    slot(0, 0, wait_w=False)
    slot(1, 1, wait_w=False)
    slot(2, 2)

    @pl.loop(1, 12)
    def _(i):
        c = 3 * i
        slot(c, 0)
        slot(c + 1, 1)
        slot(c + 2, 2)

    slot(36, 0)
    off_t = base + NFULL * CH
    pltpu.async_copy(ei_hbm.at[0, pl.ds(off_t, TAIL)], rt, semi[0])
    pltpu.async_copy(ei_hbm.at[1, pl.ds(off_t, TAIL)], ct, semi[0])
    slot(37, 1, idx_next=False)
    slot(38, 2, gather_next=False, idx_next=False, tail_work=True)
    pltpu.make_async_copy(pos16_hbm.at[rt], grt, semg[0]).wait()
    pltpu.make_async_copy(pos16_hbm.at[ct], gct, semg[0]).wait()
    pltpu.make_async_copy(f1_hbm.at[ct], gft, semg[0]).wait()
    _geom_chunk(TAIL, grt, gct, st)
    pltpu.async_copy(st, s_out.at[pl.ds(off_t, TAIL)], semw[0])
    pltpu.async_copy(gft, f1c_out.at[pl.ds(off_t, TAIL)], semw[0])
    wait_writes(1)
    wait_writes(2)
    pltpu.make_async_copy(st, s_out.at[pl.ds(0, TAIL)], semw[0]).wait()
    pltpu.make_async_copy(gft, f1c_out.at[pl.ds(0, TAIL)], semw[0]).wait()


def _sc_gather(pos16, f1, ei):
    mesh = plsc.VectorSubcoreMesh(core_axis_name="c", subcore_axis_name="s",
                                  num_cores=NC, num_subcores=NS)
    idx3 = [pltpu.VMEM((IDXV,), jnp.int32)] * 12
    big3 = ([pltpu.VMEM((CH, POS_W), _F32)] * 6
            + [pltpu.VMEM((CH, HID), _F32)] * 3
            + [pltpu.VMEM((CH, S_W), _F32)] * 3)
    tails = [pltpu.VMEM((TAIL,), jnp.int32)] * 2 + [
        pltpu.VMEM((TAIL, POS_W), _F32),
        pltpu.VMEM((TAIL, POS_W), _F32),
        pltpu.VMEM((TAIL, HID), _F32),
        pltpu.VMEM((TAIL, S_W), _F32),
    ]
    fn = pl.kernel(
        _gather_kernel_body,
        out_type=(jax.ShapeDtypeStruct((N_EDGES, S_W), _F32),
                  jax.ShapeDtypeStruct((N_EDGES, HID), _F32)),
        mesh=mesh,
        scratch_types=idx3 + big3 + tails + [pltpu.SemaphoreType.DMA] * 9,
        compiler_params=pltpu.CompilerParams(use_tc_tiling_on_sc=False,
                                             needs_layout_passes=False),
    )
    return fn(pos16, f1, ei)


def _mlp_body(s16_ref, f1hat_ref, mhat16_ref, w2hat_ref, b2hat_ref, out_ref):
    u16 = jnp.dot(s16_ref[...], mhat16_ref[...], preferred_element_type=_F32)
    u = u16.reshape(out_ref.shape) + f1hat_ref[...]
    h1 = _silu(u)
    h2 = _silu(jnp.dot(h1, w2hat_ref[...], preferred_element_type=_F32)
               + b2hat_ref[...])
    out_ref[...] = h2


def _mlp(s16, f1hat, Mhat16, W2hat, b2hat):
    rows = N_EDGES // PACK
    blk = 8000
    grid = rows // blk
    return pl.pallas_call(
        _mlp_body,
        grid=(grid,),
        in_specs=[
            pl.BlockSpec((blk // 4, 16 * S_W), lambda i: (i, 0)),
            pl.BlockSpec((blk, PACK * HID), lambda i: (i, 0)),
            pl.BlockSpec((16 * S_W, 16 * HID), lambda i: (0, 0)),
            pl.BlockSpec((PACK * HID, PACK * HID), lambda i: (0, 0)),
            pl.BlockSpec((1, PACK * HID), lambda i: (0, 0)),
        ],
        out_specs=pl.BlockSpec((blk, PACK * HID), lambda i: (i, 0)),
        out_shape=jax.ShapeDtypeStruct((rows, PACK * HID), _F32),
    )(s16, f1hat, Mhat16, W2hat, b2hat)


def _scatter_kernel_body(h2_hbm, ei_hbm, zeros_h_hbm, zeros_d_hbm, degrows_hbm,
                         ph_out, pd_out, *scr):
    rA = scr[0:4]
    rB = scr[4:8]
    mb = scr[8:12]
    degbuf = scr[12]
    rt, mt = scr[13:15]
    acc_h, acc_d = scr[15:17]
    seml = scr[17:21]
    sema = scr[21:25]
    semt = scr[25]

    c_ax = lax.axis_index("c")
    s_ax = lax.axis_index("s")
    wid = s_ax * NC + c_ax
    stripe = pl.ds(s_ax * ROWS_PER_SUB, ROWS_PER_SUB)
    pltpu.sync_copy(zeros_h_hbm.at[stripe], acc_h.at[stripe])
    pltpu.sync_copy(zeros_d_hbm.at[stripe], acc_d.at[stripe])
    pltpu.sync_copy(degrows_hbm, degbuf)
    plsc.subcore_barrier()

    base = wid * E_PER_W

    def issue_loads(b, c):
        off = base + c * CH
        pltpu.async_copy(ei_hbm.at[0, pl.ds(off, IDXV)], rA[b], seml[b])
        pltpu.async_copy(ei_hbm.at[0, pl.ds(off + IDXV, IDXV)], rB[b], seml[b])
        pltpu.async_copy(h2_hbm.at[pl.ds(off, CH)], mb[b], seml[b])

    def wait_loads(b):
        pltpu.make_async_copy(ei_hbm.at[0, pl.ds(0, IDXV)], rA[b], seml[b]).wait()
        pltpu.make_async_copy(ei_hbm.at[0, pl.ds(0, IDXV)], rB[b], seml[b]).wait()
        pltpu.make_async_copy(h2_hbm.at[pl.ds(0, CH)], mb[b], seml[b]).wait()

    def issue_adds(b):
        pltpu.async_copy(mb[b].at[pl.ds(0, IDXV)], acc_h.at[rA[b]], sema[b],
                         add=True)
        pltpu.async_copy(mb[b].at[pl.ds(IDXV, IDXV)], acc_h.at[rB[b]], sema[b],
                         add=True)
        pltpu.async_copy(degbuf, acc_d.at[rA[b]], sema[b], add=True)
        pltpu.async_copy(degbuf, acc_d.at[rB[b]], sema[b], add=True)

    def wait_adds(b):
        pltpu.make_async_copy(mb[b].at[pl.ds(0, IDXV)], acc_h.at[rA[b]],
                              sema[b]).wait()
        pltpu.make_async_copy(mb[b].at[pl.ds(IDXV, IDXV)], acc_h.at[rB[b]],
                              sema[b]).wait()
        pltpu.make_async_copy(degbuf, acc_d.at[rA[b]], sema[b]).wait()
        pltpu.make_async_copy(degbuf, acc_d.at[rB[b]], sema[b]).wait()

    def slot(c, b, *, wait_a=True, loads_next=True):
        wait_loads(b)
        issue_adds(b)
        if wait_a:
            wait_adds((b + 2) % 4)
        if loads_next:
            issue_loads((b + 2) % 4, c + 2)

    issue_loads(0, 0)
    issue_loads(1, 1)
    slot(0, 0, wait_a=False)
    slot(1, 1, wait_a=False)

    @pl.loop(0, 8)
    def _(i):
        c = 2 + 4 * i
        slot(c, 2)
        slot(c + 1, 3)
        slot(c + 2, 0)
        slot(c + 3, 1)

    slot(34, 2)
    slot(35, 3)
    slot(36, 0)
    slot(37, 1, loads_next=False)
    slot(38, 2, loads_next=False)
    off_t = base + NFULL * CH
    pltpu.async_copy(ei_hbm.at[0, pl.ds(off_t, TAIL)], rt, semt)
    pltpu.async_copy(h2_hbm.at[pl.ds(off_t, TAIL)], mt, semt)
    wait_adds(1)
    pltpu.make_async_copy(ei_hbm.at[0, pl.ds(0, TAIL)], rt, semt).wait()
    pltpu.make_async_copy(h2_hbm.at[pl.ds(0, TAIL)], mt, semt).wait()
    pltpu.async_copy(mt, acc_h.at[rt], semt, add=True)
    pltpu.async_copy(degbuf.at[pl.ds(0, TAIL)], acc_d.at[rt], semt, add=True)
    wait_adds(2)
    pltpu.make_async_copy(mt, acc_h.at[rt], semt).wait()
    pltpu.make_async_copy(degbuf.at[pl.ds(0, TAIL)], acc_d.at[rt], semt).wait()

    plsc.subcore_barrier()
    pltpu.sync_copy(acc_h.at[stripe], ph_out.at[c_ax].at[stripe])
    pltpu.sync_copy(acc_d.at[stripe], pd_out.at[c_ax].at[stripe])


def _sc_scatter(h2, ei, zeros_h, zeros_d, degrows):
    mesh = plsc.VectorSubcoreMesh(core_axis_name="c", subcore_axis_name="s",
                                  num_cores=NC, num_subcores=NS)
    fn = pl.kernel(
        _scatter_kernel_body,
        out_type=(jax.ShapeDtypeStruct((NC, N_NODES, HID), _F32),
                  jax.ShapeDtypeStruct((NC, N_NODES, DEG_W), _F32)),
        mesh=mesh,
        scratch_types=(
            [pltpu.VMEM((IDXV,), jnp.int32)] * 8
            + [pltpu.VMEM((CH, HID), _F32)] * 4
            + [pltpu.VMEM((IDXV, DEG_W), _F32)]
            + [pltpu.VMEM((TAIL,), jnp.int32), pltpu.VMEM((TAIL, HID), _F32)]
            + [pltpu.VMEM_SHARED((N_NODES, HID), _F32),
               pltpu.VMEM_SHARED((N_NODES, DEG_W), _F32)]
            + [pltpu.SemaphoreType.DMA] * 9
        ),
        compiler_params=pltpu.CompilerParams(use_tc_tiling_on_sc=False),
    )
    return fn(h2, ei, zeros_h, zeros_d, degrows)


def _final_body(ph_ref, pd_ref, w3_ref, b3r_ref, out_ref):
    rows = N_NODES // 4
    p4 = ph_ref[0:rows] + ph_ref[rows:2 * rows]
    d4 = pd_ref[0:rows] + pd_ref[rows:2 * rows]
    w3s = w3_ref[:, 0:OUT_C]
    for k in range(1, N_SH):
        w3s = w3s + w3_ref[:, k * OUT_C:(k + 1) * OUT_C]
    b3s = jnp.sum(b3r_ref[...], axis=0, keepdims=True)
    cols = []
    for j in range(4):
        cols.append(jnp.dot(p4[:, j * HID:(j + 1) * HID], w3s,
                            preferred_element_type=_F32)
                    + d4[:, j * HID:j * HID + 1] * b3s)
    out4 = jnp.concatenate(cols, axis=1)
    out_ref[...] = out4.reshape(N_NODES, OUT_C)


def _final(ph2, pd2, W3, b3r16):
    rows = N_NODES // 4
    return pl.pallas_call(
        _final_body,
        grid=(1,),
        in_specs=[
            pl.BlockSpec((NC * rows, 4 * HID), lambda i: (0, 0)),
            pl.BlockSpec((NC * rows, 4 * DEG_W), lambda i: (0, 0)),
            pl.BlockSpec((HID, N_SH * OUT_C), lambda i: (0, 0)),
            pl.BlockSpec((16, OUT_C), lambda i: (0, 0)),
        ],
        out_specs=pl.BlockSpec((N_NODES, OUT_C), lambda i: (0, 0)),
        out_shape=jax.ShapeDtypeStruct((N_NODES, OUT_C), _F32),
    )(ph2, pd2, W3, b3r16)


_SH_C0 = 0.5 / math.sqrt(math.pi)
_SH_C1 = 0.5 * math.sqrt(3.0 / (2.0 * math.pi))
_SH_C2 = 0.5 * math.sqrt(3.0 / math.pi)
_SH_C4 = 0.25 * math.sqrt(15.0 / (2.0 * math.pi))
_SH_C5 = 0.5 * math.sqrt(15.0 / (2.0 * math.pi))
_SH_C6 = 0.25 * math.sqrt(5.0 / math.pi)


def _blockdiag(m, k):
    r, c = m.shape
    e = jnp.eye(k, dtype=m.dtype)
    return (e[:, None, :, None] * m[None, :, None, :]).reshape(k * r, k * c)


def kernel(features, positions, edge_index, W1, b1, W2, b2, W3, b3):
    ei = edge_index.astype(jnp.int32)

    W1a = W1[:N_SH]
    W1b = W1[N_SH:]
    M = jnp.stack([
        _SH_C1 * (W1a[3] - W1a[1]),
        jnp.zeros((HID,), _F32),
        _SH_C2 * W1a[2],
        _SH_C4 * (W1a[4] + W1a[8]) - _SH_C6 * W1a[6],
        -_SH_C4 * (W1a[4] + W1a[8]) - _SH_C6 * W1a[6],
        2.0 * _SH_C6 * W1a[6],
        _SH_C5 * (W1a[7] - W1a[5]),
        jnp.zeros((HID,), _F32),
    ])
    b1e = b1 + _SH_C0 * W1a[0]
    W1b4 = _blockdiag(W1b, 4)
    b1hat = jnp.tile(b1e, (4,)).reshape(1, 4 * HID)
    Mhat16 = _blockdiag(M, 16)
    W2hat = _blockdiag(W2, PACK)
    b2hat = jnp.tile(b2, (PACK,)).reshape(1, PACK * HID)
    b3r16 = jnp.pad(b3.reshape(N_SH, OUT_C), ((0, 16 - N_SH), (0, 0)))
    zeros_h = jnp.zeros((N_NODES, HID), _F32)
    zeros_d = jnp.zeros((N_NODES, DEG_W), _F32)
    degrows = jnp.zeros((IDXV, DEG_W), _F32).at[:, 0].set(1.0)

    feat4 = features.reshape(N_NODES // 4, 4 * IN_C)
    f1hat4 = _prep(feat4, W1b4, b1hat)
    f1 = f1hat4.reshape(N_NODES, HID)
    pos16 = jnp.pad(positions.astype(_F32), ((0, 0), (0, POS_W - 3)))
    s_mat, f1c = _sc_gather(pos16, f1, ei)
    s16 = s_mat.reshape(N_EDGES // 16, 16 * S_W)
    f1hat = f1c.reshape(N_EDGES // PACK, PACK * HID)
    h2hat = _mlp(s16, f1hat, Mhat16, W2hat, b2hat)
    h2 = h2hat.reshape(N_EDGES, HID)
    ph, pd = _sc_scatter(h2, ei, zeros_h, zeros_d, degrows)
    ph2 = ph.reshape(NC * N_NODES // 4, 4 * HID)
    pd2 = pd.reshape(NC * N_NODES // 4, 4 * DEG_W)
    return _final(ph2, pd2, W3, b3r16)
